# Initial kernel scaffold; baseline (speedup 1.0000x reference)
import jax
import jax.numpy as jnp
from jax.experimental import pallas as pl


def kernel(x, edge_index, edge_weight, W_enc1, b_enc1, W_enc2, b_enc2, W_enc3, b_enc3, W_z, b_z, W_dec1, b_dec1, W_dec2, b_dec2, W_dec3, b_dec3, W_xde, b_xde, W_c1, b_c1, W_c2, b_c2):
    raise NotImplementedError("write your pallas kernel here")


# speedup vs baseline: 1.0000x; 1.0000x over previous
import jax
import jax.numpy as jnp
from jax.experimental import pallas as pl

ALPHA = 0.5


def _norm(src, dst, ew, n):
    loop = jnp.arange(n, dtype=src.dtype)
    src2 = jnp.concatenate([src, loop])
    dst2 = jnp.concatenate([dst, loop])
    ew2 = jnp.concatenate([ew, jnp.ones((n,), ew.dtype)])
    deg = jax.ops.segment_sum(ew2, dst2, num_segments=n)
    dinv = jnp.where(deg > 0, jax.lax.rsqrt(jnp.maximum(deg, 1e-12)), 0.0)
    return src2, dst2, dinv[src2] * ew2 * dinv[dst2]


def _gcn(x, W, b, src2, dst2, nrm, n):
    h = x @ W
    return jax.ops.segment_sum(nrm[:, None] * h[src2], dst2, num_segments=n) + b


def _cos(a, b):
    num = jnp.sum(a * b, axis=1)
    den = jnp.maximum(jnp.linalg.norm(a, axis=1), 1e-8) * jnp.maximum(
        jnp.linalg.norm(b, axis=1), 1e-8)
    return num / den


def _softmax_pair_kernel(a_ref, b_ref, oa_ref, ob_ref):
    a = a_ref[...]
    b = b_ref[...]
    oa_ref[...] = jax.nn.softmax(a, axis=1)
    ob_ref[...] = jax.nn.softmax(b, axis=1)


def _softmax_pair(a, b):
    return pl.pallas_call(
        _softmax_pair_kernel,
        out_shape=(jax.ShapeDtypeStruct(a.shape, a.dtype),
                   jax.ShapeDtypeStruct(b.shape, b.dtype)),
    )(a, b)


def kernel(x, edge_index, edge_weight, W_enc1, b_enc1, W_enc2, b_enc2, W_enc3,
           b_enc3, W_z, b_z, W_dec1, b_dec1, W_dec2, b_dec2, W_dec3, b_dec3,
           W_xde, b_xde, W_c1, b_c1, W_c2, b_c2):
    n = x.shape[0]
    src = edge_index[0]
    dst = edge_index[1]
    ones_e = jnp.ones((src.shape[0],), jnp.float32)
    s1, d1, nrm1 = _norm(src, dst, ones_e, n)
    s2, d2, nrm2 = _norm(src, dst, edge_weight, n)
    enc_h1 = jax.nn.relu(_gcn(x, W_enc1, b_enc1, s1, d1, nrm1, n))
    enc_h2 = jax.nn.relu(_gcn(enc_h1, W_enc2, b_enc2, s1, d1, nrm1, n))
    enc_h3 = jax.nn.relu(_gcn(enc_h2, W_enc3, b_enc3, s1, d1, nrm1, n))
    z_en = _gcn(enc_h3, W_z, b_z, s1, d1, nrm1, n)
    dec_h1 = jax.nn.relu(_gcn(z_en, W_dec1, b_dec1, s1, d1, nrm1, n))
    dec_h2 = jax.nn.relu(_gcn(dec_h1, W_dec2, b_dec2, s1, d1, nrm1, n))
    dec_h3 = jax.nn.relu(_gcn(dec_h2, W_dec3, b_dec3, s1, d1, nrm1, n))
    x_de = _gcn(dec_h3, W_xde, b_xde, s1, d1, nrm1, n)
    x_out = jax.nn.sigmoid(_gcn(x, W_c1, b_c1, s2, d2, nrm2, n))
    x_in = x_out
    layers = []
    for _ in range(3):
        xo = jax.ops.segment_sum(nrm1[:, None] * x_in[s1], d1, num_segments=n)
        w = _cos(xo, x_in)
        x_result = w[:, None] * xo
        x_in = jax.nn.relu(ALPHA * x_result + x_in)
        layers.append(x_in)
    x_all = jnp.mean(jnp.stack(layers, axis=1), axis=1)
    x_out2 = jnp.tanh(_gcn(x_all, W_c2, b_c2, s2, d2, nrm2, n))
    o, q = _softmax_pair(x_out2, z_en)
    return (x_out2, x_de, z_en, q, o)


# speedup vs baseline: 9.8906x; 9.8902x over previous
import functools

import jax
import jax.numpy as jnp
from jax import lax
from jax.experimental import pallas as pl
from jax.experimental.pallas import tpu as pltpu
from jax.experimental.pallas import tpu_sc as plsc

N = 10000
E = 320000
NCORE = 2
NSUB = 16
NW = NCORE * NSUB
EPT = E // NW
K = 80
SUBC = 5
SUP = EPT // (SUBC * K)
RPT = 624
SCP = 48
TAIL = N - NSUB * RPT
TAIL_BASE = NSUB * RPT
D = 128

_MESH = plsc.VectorSubcoreMesh(core_axis_name="c", subcore_axis_name="s")


def _make_spmm(weighted, sw=0):
    scratch = [
        pltpu.VMEM_SHARED((N, D), jnp.float32),
        pltpu.VMEM((SUBC, K), jnp.int32),
        pltpu.VMEM((SUBC, K), jnp.int32),
        pltpu.VMEM((K, D), jnp.float32),
        pltpu.VMEM((SCP, D), jnp.float32),
        pltpu.SemaphoreType.DMA,
    ]
    if weighted:
        scratch.append(pltpu.VMEM((SUBC, K), jnp.float32))

    @functools.partial(
        pl.kernel,
        mesh=_MESH,
        out_type=jax.ShapeDtypeStruct((NCORE, N, D), jnp.float32),
        scratch_types=scratch,
    )
    def spmm(*refs):
        if weighted:
            h, srcr, dstr, wr, out, acc, isrc, idst, rows, zbuf, sem, wbuf = refs
        else:
            h, srcr, dstr, out, acc, isrc, idst, rows, zbuf, sem = refs
        c = lax.axis_index("c")
        s = lax.axis_index("s")
        wid = s * NCORE + c
        z16 = jnp.zeros((16,), jnp.float32)

        def zrow(r, carry):
            for j in range(D // 16):
                zbuf[r, pl.ds(16 * j, 16)] = z16
            return carry

        lax.fori_loop(0, SCP, zrow, 0)
        base = s * RPT

        def zcp(t, carry):
            off = pl.multiple_of(base + t * SCP, 8)
            pltpu.sync_copy(zbuf, acc.at[pl.ds(off, SCP)])
            return carry

        lax.fori_loop(0, RPT // SCP, zcp, 0)

        @pl.when(s == NSUB - 1)
        def _():
            pltpu.sync_copy(zbuf.at[pl.ds(0, TAIL)],
                            acc.at[pl.ds(TAIL_BASE, TAIL)])

        plsc.subcore_barrier()

        def outer(t, carry):
            pltpu.sync_copy(srcr.at[wid, t], isrc)
            pltpu.sync_copy(dstr.at[wid, t], idst)
            if weighted:
                pltpu.sync_copy(wr.at[wid, t], wbuf)
            for j in range(SUBC):
                pltpu.async_copy(h.at[isrc.at[j]], rows, sem).wait()
                if weighted:
                    def scale(g, carry2):
                        st = pl.multiple_of(g * 16, 8)
                        w16 = wbuf[j, pl.ds(st, 16)]
                        for b in range(16):
                            wv = w16[b]
                            r = g * 16 + b
                            for q in range(sw // 16):
                                sl = pl.ds(16 * q, 16)
                                rows[r, sl] = rows[r, sl] * wv
                        return carry2

                    lax.fori_loop(0, K // 16, scale, 0)
                pltpu.sync_copy(rows, acc.at[idst.at[j]], add=True)
            return carry

        lax.fori_loop(0, SUP, outer, 0)
        plsc.subcore_barrier()

        def wcp(t, carry):
            off = pl.multiple_of(base + t * SCP, 8)
            pltpu.sync_copy(acc.at[pl.ds(off, SCP)],
                            out.at[c, pl.ds(off, SCP)])
            return carry

        lax.fori_loop(0, RPT // SCP, wcp, 0)

        @pl.when(s == NSUB - 1)
        def _():
            pltpu.sync_copy(acc.at[pl.ds(TAIL_BASE, TAIL)],
                            out.at[c, pl.ds(TAIL_BASE, TAIL)])

    return spmm


def _make_deg():
    @functools.partial(
        pl.kernel,
        mesh=_MESH,
        out_type=jax.ShapeDtypeStruct((NCORE, N, D), jnp.float32),
        scratch_types=[
            pltpu.VMEM_SHARED((N, D), jnp.float32),
            pltpu.VMEM((SUBC, K), jnp.int32),
            pltpu.VMEM((SUBC, K), jnp.float32),
            pltpu.VMEM((K, D), jnp.float32),
            pltpu.VMEM((SCP, D), jnp.float32),
        ],
    )
    def deg(dstr, wr, out, acc, idst, wbuf, rows, zbuf):
        c = lax.axis_index("c")
        s = lax.axis_index("s")
        wid = s * NCORE + c
        z16 = jnp.zeros((16,), jnp.float32)
        one16 = jnp.ones((16,), jnp.float32)

        def zrow(r, carry):
            for j in range(D // 16):
                zbuf[r, pl.ds(16 * j, 16)] = z16
            return carry

        lax.fori_loop(0, SCP, zrow, 0)

        def orow(r, carry):
            rows[r, pl.ds(0, 16)] = one16
            for j in range(2, D // 16):
                rows[r, pl.ds(16 * j, 16)] = z16
            return carry

        lax.fori_loop(0, K, orow, 0)
        base = s * RPT

        def zcp(t, carry):
            off = pl.multiple_of(base + t * SCP, 8)
            pltpu.sync_copy(zbuf, acc.at[pl.ds(off, SCP)])
            return carry

        lax.fori_loop(0, RPT // SCP, zcp, 0)

        @pl.when(s == NSUB - 1)
        def _():
            pltpu.sync_copy(zbuf.at[pl.ds(0, TAIL)],
                            acc.at[pl.ds(TAIL_BASE, TAIL)])

        plsc.subcore_barrier()

        def outer(t, carry):
            pltpu.sync_copy(dstr.at[wid, t], idst)
            pltpu.sync_copy(wr.at[wid, t], wbuf)
            for j in range(SUBC):
                def fill(g, carry2):
                    st = pl.multiple_of(g * 16, 8)
                    w16 = wbuf[j, pl.ds(st, 16)]
                    for b in range(16):
                        rows[g * 16 + b, pl.ds(16, 16)] = one16 * w16[b]
                    return carry2

                lax.fori_loop(0, K // 16, fill, 0)
                pltpu.sync_copy(rows, acc.at[idst.at[j]], add=True)
            return carry

        lax.fori_loop(0, SUP, outer, 0)
        plsc.subcore_barrier()

        def wcp(t, carry):
            off = pl.multiple_of(base + t * SCP, 8)
            pltpu.sync_copy(acc.at[pl.ds(off, SCP)],
                            out.at[c, pl.ds(off, SCP)])
            return carry

        lax.fori_loop(0, RPT // SCP, wcp, 0)

        @pl.when(s == NSUB - 1)
        def _():
            pltpu.sync_copy(acc.at[pl.ds(TAIL_BASE, TAIL)],
                            out.at[c, pl.ds(TAIL_BASE, TAIL)])

    return deg


_spmm = _make_spmm(False)
_spmm_w64 = _make_spmm(True, 64)
_spmm_w16 = _make_spmm(True, 16)
_deg = _make_deg()


BR = 1000
GRID = N // BR


def _row_spec(d):
    return pl.BlockSpec((BR, d), lambda i: (i, 0))


def _full_spec(a, b):
    return pl.BlockSpec((a, b), lambda i: (0, 0))


def _pad128(a):
    fo = a.shape[1]
    if fo == D:
        return a
    return jnp.concatenate(
        [a, jnp.zeros((a.shape[0], D - fo), jnp.float32)], axis=1)


def _mm_scale(x, W, dinv, scale=1.0):
    fi, fo = W.shape

    def body(x_ref, w_ref, dv_ref, o_ref):
        acc = jnp.dot(x_ref[...], w_ref[...],
                      preferred_element_type=jnp.float32)
        o_ref[...] = _pad128((dv_ref[...] * scale) * acc)

    return pl.pallas_call(
        body,
        grid=(GRID,),
        in_specs=[_row_spec(fi), _full_spec(fi, fo), _row_spec(1)],
        out_specs=_row_spec(D),
        out_shape=jax.ShapeDtypeStruct((N, D), jnp.float32),
    )(x, W, dinv)


def _combine(p0, p1, hp, d, dinv, b, act, W_next=None, post_dinv=None):
    if W_next is not None:
        def body(p0_ref, p1_ref, hp_ref, dv_ref, b_ref, w_ref, pdv_ref,
                 a_ref, h_ref):
            ps = (p0_ref[...] + p1_ref[...] + hp_ref[...])[:, :d]
            a = act(dv_ref[...] * ps + b_ref[...])
            a_ref[...] = a
            h_ref[...] = _pad128(pdv_ref[...] * jnp.dot(
                a, w_ref[...], preferred_element_type=jnp.float32))

        fo = W_next.shape[1]
        return pl.pallas_call(
            body,
            grid=(GRID,),
            in_specs=[_row_spec(D), _row_spec(D), _row_spec(D), _row_spec(1),
                      _full_spec(1, d), _full_spec(d, fo), _row_spec(1)],
            out_specs=[_row_spec(d), _row_spec(D)],
            out_shape=[jax.ShapeDtypeStruct((N, d), jnp.float32),
                       jax.ShapeDtypeStruct((N, D), jnp.float32)],
        )(p0, p1, hp, dinv, b, W_next, post_dinv)

    if post_dinv is not None:
        def body(p0_ref, p1_ref, hp_ref, dv_ref, b_ref, pdv_ref,
                 a_ref, u_ref):
            ps = (p0_ref[...] + p1_ref[...] + hp_ref[...])[:, :d]
            a = act(dv_ref[...] * ps + b_ref[...])
            a_ref[...] = a
            u_ref[...] = _pad128(pdv_ref[...] * a)

        return pl.pallas_call(
            body,
            grid=(GRID,),
            in_specs=[_row_spec(D), _row_spec(D), _row_spec(D), _row_spec(1),
                      _full_spec(1, d), _row_spec(1)],
            out_specs=[_row_spec(d), _row_spec(D)],
            out_shape=[jax.ShapeDtypeStruct((N, d), jnp.float32),
                       jax.ShapeDtypeStruct((N, D), jnp.float32)],
        )(p0, p1, hp, dinv, b, post_dinv)

    def body(p0_ref, p1_ref, hp_ref, dv_ref, b_ref, a_ref):
        ps = (p0_ref[...] + p1_ref[...] + hp_ref[...])[:, :d]
        a_ref[...] = act(dv_ref[...] * ps + b_ref[...])

    return pl.pallas_call(
        body,
        grid=(GRID,),
        in_specs=[_row_spec(D), _row_spec(D), _row_spec(D), _row_spec(1),
                  _full_spec(1, d)],
        out_specs=_row_spec(d),
        out_shape=jax.ShapeDtypeStruct((N, d), jnp.float32),
    )(p0, p1, hp, dinv, b)


def _deg_combine(p0, p1):
    def body(p0_ref, p1_ref, d1_ref, d2_ref):
        ps = p0_ref[...] + p1_ref[...]
        deg1 = ps[:, 0:1] + 1.0
        deg2 = ps[:, 16:17] + 1.0
        d1_ref[...] = jnp.where(
            deg1 > 0, lax.rsqrt(jnp.maximum(deg1, 1e-12)), 0.0)
        d2_ref[...] = jnp.where(
            deg2 > 0, lax.rsqrt(jnp.maximum(deg2, 1e-12)), 0.0)

    return pl.pallas_call(
        body,
        grid=(GRID,),
        in_specs=[_row_spec(D), _row_spec(D)],
        out_specs=[_row_spec(1), _row_spec(1)],
        out_shape=[jax.ShapeDtypeStruct((N, 1), jnp.float32),
                   jax.ShapeDtypeStruct((N, 1), jnp.float32)],
    )(p0, p1)


def _loop_step(p0, p1, u, xin, xsum, dinv):
    def body(p0_ref, p1_ref, u_ref, x_ref, xs_ref, dv_ref,
             xo_ref, uo_ref, xso_ref):
        xin_v = x_ref[...]
        ps = (p0_ref[...] + p1_ref[...] + u_ref[...])[:, :64]
        xo = dv_ref[...] * ps
        num = jnp.sum(xo * xin_v, axis=1, keepdims=True)
        na = jnp.maximum(
            jnp.sqrt(jnp.sum(xo * xo, axis=1, keepdims=True)), 1e-8)
        nb = jnp.maximum(
            jnp.sqrt(jnp.sum(xin_v * xin_v, axis=1, keepdims=True)), 1e-8)
        w = num / (na * nb)
        xnew = jax.nn.relu(0.5 * (w * xo) + xin_v)
        xo_ref[...] = xnew
        uo_ref[...] = _pad128(dv_ref[...] * xnew)
        xso_ref[...] = xs_ref[...] + xnew

    return pl.pallas_call(
        body,
        grid=(GRID,),
        in_specs=[_row_spec(D), _row_spec(D), _row_spec(D), _row_spec(64),
                  _row_spec(64), _row_spec(1)],
        out_specs=[_row_spec(64), _row_spec(D), _row_spec(64)],
        out_shape=[jax.ShapeDtypeStruct((N, 64), jnp.float32),
                   jax.ShapeDtypeStruct((N, D), jnp.float32),
                   jax.ShapeDtypeStruct((N, 64), jnp.float32)],
    )(p0, p1, u, xin, xsum, dinv)


def _final(t0, t1, g2p, dinv2, b_c2, z_en):
    def body(t0_ref, t1_ref, g_ref, dv_ref, b_ref, z_ref,
             x2_ref, o_ref, q_ref):
        ps = (t0_ref[...] + t1_ref[...] + g_ref[...])[:, :16]
        x2 = jnp.tanh(dv_ref[...] * ps + b_ref[...])
        x2_ref[...] = x2
        o_ref[...] = jax.nn.softmax(x2, axis=1)
        q_ref[...] = jax.nn.softmax(z_ref[...], axis=1)

    return pl.pallas_call(
        body,
        grid=(GRID,),
        in_specs=[_row_spec(D), _row_spec(D), _row_spec(D), _row_spec(1),
                  _full_spec(1, 16), _row_spec(16)],
        out_specs=[_row_spec(16)] * 3,
        out_shape=[jax.ShapeDtypeStruct((N, 16), jnp.float32)] * 3,
    )(t0, t1, g2p, dinv2, b_c2, z_en)


def _identity(v):
    return v


def kernel(x, edge_index, edge_weight, W_enc1, b_enc1, W_enc2, b_enc2, W_enc3,
           b_enc3, W_z, b_z, W_dec1, b_dec1, W_dec2, b_dec2, W_dec3, b_dec3,
           W_xde, b_xde, W_c1, b_c1, W_c2, b_c2):
    src = edge_index[0].astype(jnp.int32).reshape(NW, SUP, SUBC, K)
    dst = edge_index[1].astype(jnp.int32).reshape(NW, SUP, SUBC, K)
    wres = edge_weight.astype(jnp.float32).reshape(NW, SUP, SUBC, K)
    relu = jax.nn.relu

    degp = _deg(dst, wres)
    dinv1, dinv2 = _deg_combine(degp[0], degp[1])

    def spmm1(hp):
        p = _spmm(hp, src, dst)
        return p[0], p[1]

    def spmm2(hp, sw):
        f = {16: _spmm_w16, 64: _spmm_w64}[sw]
        p = f(hp, src, dst, wres)
        return p[0], p[1]

    b1 = b_enc1.reshape(1, -1)
    b2 = b_enc2.reshape(1, -1)
    b3 = b_enc3.reshape(1, -1)
    bz = b_z.reshape(1, -1)
    bd1 = b_dec1.reshape(1, -1)
    bd2 = b_dec2.reshape(1, -1)
    bd3 = b_dec3.reshape(1, -1)
    bxde = b_xde.reshape(1, -1)
    bc1 = b_c1.reshape(1, -1)
    bc2 = b_c2.reshape(1, -1)

    h1p = _mm_scale(x, W_enc1, dinv1)
    p0, p1 = spmm1(h1p)
    _, h2p = _combine(p0, p1, h1p, 64, dinv1, b1, relu, W_next=W_enc2,
                      post_dinv=dinv1)
    p0, p1 = spmm1(h2p)
    _, h3p = _combine(p0, p1, h2p, 64, dinv1, b2, relu, W_next=W_enc3,
                      post_dinv=dinv1)
    p0, p1 = spmm1(h3p)
    _, hzp = _combine(p0, p1, h3p, 64, dinv1, b3, relu, W_next=W_z,
                      post_dinv=dinv1)
    p0, p1 = spmm1(hzp)
    z_en, hd1p = _combine(p0, p1, hzp, 16, dinv1, bz, _identity,
                          W_next=W_dec1, post_dinv=dinv1)
    p0, p1 = spmm1(hd1p)
    _, hd2p = _combine(p0, p1, hd1p, 64, dinv1, bd1, relu, W_next=W_dec2,
                       post_dinv=dinv1)
    p0, p1 = spmm1(hd2p)
    _, hd3p = _combine(p0, p1, hd2p, 64, dinv1, bd2, relu, W_next=W_dec3,
                       post_dinv=dinv1)
    p0, p1 = spmm1(hd3p)
    _, hxp = _combine(p0, p1, hd3p, 64, dinv1, bd3, relu, W_next=W_xde,
                      post_dinv=dinv1)
    p0, p1 = spmm1(hxp)
    x_de = _combine(p0, p1, hxp, 128, dinv1, bxde, _identity)

    g1p = _mm_scale(x, W_c1, dinv2)
    t0, t1 = spmm2(g1p, 64)
    xin, u = _combine(t0, t1, g1p, 64, dinv2, bc1, jax.nn.sigmoid,
                      post_dinv=dinv1)
    xsum = jnp.zeros((N, 64), jnp.float32)
    for _ in range(3):
        p0, p1 = spmm1(u)
        xin, u, xsum = _loop_step(p0, p1, u, xin, xsum, dinv1)
    g2p = _mm_scale(xsum, W_c2, dinv2, scale=1.0 / 3.0)
    t0, t1 = spmm2(g2p, 16)
    x_out2, o, q = _final(t0, t1, g2p, dinv2, bc2, z_en)
    return (x_out2, x_de, z_en, q, o)


# speedup vs baseline: 12.9994x; 1.3143x over previous
import functools

import jax
import jax.numpy as jnp
from jax import lax
from jax.experimental import pallas as pl
from jax.experimental.pallas import tpu as pltpu
from jax.experimental.pallas import tpu_sc as plsc

N = 10000
E = 320000
NCORE = 2
NSUB = 16
NW = NCORE * NSUB
EPT = E // NW
K = 80
SUBC = 5
SUP = EPT // (SUBC * K)
RPT = 624
SCP = 48
TAIL = N - NSUB * RPT
TAIL_BASE = NSUB * RPT
D = 128

_MESH = plsc.VectorSubcoreMesh(core_axis_name="c", subcore_axis_name="s")


def _make_spmm(weighted, sw=0):
    scratch = [
        pltpu.VMEM_SHARED((N, D), jnp.float32),
        pltpu.VMEM((SUBC, K), jnp.int32),
        pltpu.VMEM((SUBC, K), jnp.int32),
        pltpu.VMEM((K, D), jnp.float32),
        pltpu.VMEM((K, D), jnp.float32),
        pltpu.VMEM((SCP, D), jnp.float32),
        pltpu.SemaphoreType.DMA,
        pltpu.SemaphoreType.DMA,
    ]
    if weighted:
        scratch.append(pltpu.VMEM((SUBC, K), jnp.float32))

    @functools.partial(
        pl.kernel,
        mesh=_MESH,
        out_type=jax.ShapeDtypeStruct((NCORE, N, D), jnp.float32),
        scratch_types=scratch,
    )
    def spmm(*refs):
        if weighted:
            (h, srcr, dstr, wr, out, acc, isrc, idst, rows0, rows1, zbuf,
             sem0, sem1, wbuf) = refs
        else:
            (h, srcr, dstr, out, acc, isrc, idst, rows0, rows1, zbuf,
             sem0, sem1) = refs
        bufs = (rows0, rows1)
        sems = (sem0, sem1)
        c = lax.axis_index("c")
        s = lax.axis_index("s")
        wid = s * NCORE + c
        z16 = jnp.zeros((16,), jnp.float32)

        def zrow(r, carry):
            for j in range(D // 16):
                zbuf[r, pl.ds(16 * j, 16)] = z16
            return carry

        lax.fori_loop(0, SCP, zrow, 0)
        base = s * RPT

        def zcp(t, carry):
            off = pl.multiple_of(base + t * SCP, 8)
            pltpu.sync_copy(zbuf, acc.at[pl.ds(off, SCP)])
            return carry

        lax.fori_loop(0, RPT // SCP, zcp, 0)

        @pl.when(s == NSUB - 1)
        def _():
            pltpu.sync_copy(zbuf.at[pl.ds(0, TAIL)],
                            acc.at[pl.ds(TAIL_BASE, TAIL)])

        plsc.subcore_barrier()

        def outer(t, carry):
            pltpu.sync_copy(srcr.at[wid, t], isrc)
            pltpu.sync_copy(dstr.at[wid, t], idst)
            if weighted:
                pltpu.sync_copy(wr.at[wid, t], wbuf)
            handles = [None] * SUBC
            handles[0] = pltpu.async_copy(h.at[isrc.at[0]], bufs[0], sems[0])
            for j in range(SUBC):
                if j + 1 < SUBC:
                    handles[j + 1] = pltpu.async_copy(
                        h.at[isrc.at[j + 1]], bufs[(j + 1) % 2],
                        sems[(j + 1) % 2])
                handles[j].wait()
                rows = bufs[j % 2]
                if weighted:
                    def scale(g, carry2):
                        st = pl.multiple_of(g * 16, 8)
                        w16 = wbuf[j, pl.ds(st, 16)]
                        for b in range(16):
                            wv = w16[b]
                            r = g * 16 + b
                            for q in range(sw // 16):
                                sl = pl.ds(16 * q, 16)
                                rows[r, sl] = rows[r, sl] * wv
                        return carry2

                    lax.fori_loop(0, K // 16, scale, 0)
                pltpu.sync_copy(rows, acc.at[idst.at[j]], add=True)
            return carry

        lax.fori_loop(0, SUP, outer, 0)
        plsc.subcore_barrier()

        def wcp(t, carry):
            off = pl.multiple_of(base + t * SCP, 8)
            pltpu.sync_copy(acc.at[pl.ds(off, SCP)],
                            out.at[c, pl.ds(off, SCP)])
            return carry

        lax.fori_loop(0, RPT // SCP, wcp, 0)

        @pl.when(s == NSUB - 1)
        def _():
            pltpu.sync_copy(acc.at[pl.ds(TAIL_BASE, TAIL)],
                            out.at[c, pl.ds(TAIL_BASE, TAIL)])

    return spmm


def _make_deg():
    @functools.partial(
        pl.kernel,
        mesh=_MESH,
        out_type=jax.ShapeDtypeStruct((NCORE, N, D), jnp.float32),
        scratch_types=[
            pltpu.VMEM_SHARED((N, D), jnp.float32),
            pltpu.VMEM((SUBC, K), jnp.int32),
            pltpu.VMEM((SUBC, K), jnp.float32),
            pltpu.VMEM((K, D), jnp.float32),
            pltpu.VMEM((SCP, D), jnp.float32),
        ],
    )
    def deg(dstr, wr, out, acc, idst, wbuf, rows, zbuf):
        c = lax.axis_index("c")
        s = lax.axis_index("s")
        wid = s * NCORE + c
        z16 = jnp.zeros((16,), jnp.float32)
        one16 = jnp.ones((16,), jnp.float32)

        def zrow(r, carry):
            for j in range(D // 16):
                zbuf[r, pl.ds(16 * j, 16)] = z16
            return carry

        lax.fori_loop(0, SCP, zrow, 0)

        def orow(r, carry):
            rows[r, pl.ds(0, 16)] = one16
            for j in range(2, D // 16):
                rows[r, pl.ds(16 * j, 16)] = z16
            return carry

        lax.fori_loop(0, K, orow, 0)
        base = s * RPT

        def zcp(t, carry):
            off = pl.multiple_of(base + t * SCP, 8)
            pltpu.sync_copy(zbuf, acc.at[pl.ds(off, SCP)])
            return carry

        lax.fori_loop(0, RPT // SCP, zcp, 0)

        @pl.when(s == NSUB - 1)
        def _():
            pltpu.sync_copy(zbuf.at[pl.ds(0, TAIL)],
                            acc.at[pl.ds(TAIL_BASE, TAIL)])

        plsc.subcore_barrier()

        def outer(t, carry):
            pltpu.sync_copy(dstr.at[wid, t], idst)
            pltpu.sync_copy(wr.at[wid, t], wbuf)
            for j in range(SUBC):
                def fill(g, carry2):
                    st = pl.multiple_of(g * 16, 8)
                    w16 = wbuf[j, pl.ds(st, 16)]
                    for b in range(16):
                        rows[g * 16 + b, pl.ds(16, 16)] = one16 * w16[b]
                    return carry2

                lax.fori_loop(0, K // 16, fill, 0)
                pltpu.sync_copy(rows, acc.at[idst.at[j]], add=True)
            return carry

        lax.fori_loop(0, SUP, outer, 0)
        plsc.subcore_barrier()

        def wcp(t, carry):
            off = pl.multiple_of(base + t * SCP, 8)
            pltpu.sync_copy(acc.at[pl.ds(off, SCP)],
                            out.at[c, pl.ds(off, SCP)])
            return carry

        lax.fori_loop(0, RPT // SCP, wcp, 0)

        @pl.when(s == NSUB - 1)
        def _():
            pltpu.sync_copy(acc.at[pl.ds(TAIL_BASE, TAIL)],
                            out.at[c, pl.ds(TAIL_BASE, TAIL)])

    return deg


_spmm = _make_spmm(False)
_spmm_w64 = _make_spmm(True, 64)
_spmm_w16 = _make_spmm(True, 16)
_deg = _make_deg()


BR = 1000
GRID = N // BR


def _row_spec(d):
    return pl.BlockSpec((BR, d), lambda i: (i, 0))


def _full_spec(a, b):
    return pl.BlockSpec((a, b), lambda i: (0, 0))


def _pad128(a):
    fo = a.shape[1]
    if fo == D:
        return a
    return jnp.concatenate(
        [a, jnp.zeros((a.shape[0], D - fo), jnp.float32)], axis=1)


def _mm_scale(x, W, dinv, scale=1.0):
    fi, fo = W.shape

    def body(x_ref, w_ref, dv_ref, o_ref):
        acc = jnp.dot(x_ref[...], w_ref[...],
                      preferred_element_type=jnp.float32)
        o_ref[...] = _pad128((dv_ref[...] * scale) * acc)

    return pl.pallas_call(
        body,
        grid=(GRID,),
        in_specs=[_row_spec(fi), _full_spec(fi, fo), _row_spec(1)],
        out_specs=_row_spec(D),
        out_shape=jax.ShapeDtypeStruct((N, D), jnp.float32),
    )(x, W, dinv)


def _combine(p0, p1, hp, d, dinv, b, act, W_next=None, post_dinv=None):
    if W_next is not None:
        def body(p0_ref, p1_ref, hp_ref, dv_ref, b_ref, w_ref, pdv_ref,
                 a_ref, h_ref):
            ps = (p0_ref[...] + p1_ref[...] + hp_ref[...])[:, :d]
            a = act(dv_ref[...] * ps + b_ref[...])
            a_ref[...] = a
            h_ref[...] = _pad128(pdv_ref[...] * jnp.dot(
                a, w_ref[...], preferred_element_type=jnp.float32))

        fo = W_next.shape[1]
        return pl.pallas_call(
            body,
            grid=(GRID,),
            in_specs=[_row_spec(D), _row_spec(D), _row_spec(D), _row_spec(1),
                      _full_spec(1, d), _full_spec(d, fo), _row_spec(1)],
            out_specs=[_row_spec(d), _row_spec(D)],
            out_shape=[jax.ShapeDtypeStruct((N, d), jnp.float32),
                       jax.ShapeDtypeStruct((N, D), jnp.float32)],
        )(p0, p1, hp, dinv, b, W_next, post_dinv)

    if post_dinv is not None:
        def body(p0_ref, p1_ref, hp_ref, dv_ref, b_ref, pdv_ref,
                 a_ref, u_ref):
            ps = (p0_ref[...] + p1_ref[...] + hp_ref[...])[:, :d]
            a = act(dv_ref[...] * ps + b_ref[...])
            a_ref[...] = a
            u_ref[...] = _pad128(pdv_ref[...] * a)

        return pl.pallas_call(
            body,
            grid=(GRID,),
            in_specs=[_row_spec(D), _row_spec(D), _row_spec(D), _row_spec(1),
                      _full_spec(1, d), _row_spec(1)],
            out_specs=[_row_spec(d), _row_spec(D)],
            out_shape=[jax.ShapeDtypeStruct((N, d), jnp.float32),
                       jax.ShapeDtypeStruct((N, D), jnp.float32)],
        )(p0, p1, hp, dinv, b, post_dinv)

    def body(p0_ref, p1_ref, hp_ref, dv_ref, b_ref, a_ref):
        ps = (p0_ref[...] + p1_ref[...] + hp_ref[...])[:, :d]
        a_ref[...] = act(dv_ref[...] * ps + b_ref[...])

    return pl.pallas_call(
        body,
        grid=(GRID,),
        in_specs=[_row_spec(D), _row_spec(D), _row_spec(D), _row_spec(1),
                  _full_spec(1, d)],
        out_specs=_row_spec(d),
        out_shape=jax.ShapeDtypeStruct((N, d), jnp.float32),
    )(p0, p1, hp, dinv, b)


def _deg_combine(p0, p1):
    def body(p0_ref, p1_ref, d1_ref, d2_ref):
        ps = p0_ref[...] + p1_ref[...]
        deg1 = ps[:, 0:1] + 1.0
        deg2 = ps[:, 16:17] + 1.0
        d1_ref[...] = jnp.where(
            deg1 > 0, lax.rsqrt(jnp.maximum(deg1, 1e-12)), 0.0)
        d2_ref[...] = jnp.where(
            deg2 > 0, lax.rsqrt(jnp.maximum(deg2, 1e-12)), 0.0)

    return pl.pallas_call(
        body,
        grid=(GRID,),
        in_specs=[_row_spec(D), _row_spec(D)],
        out_specs=[_row_spec(1), _row_spec(1)],
        out_shape=[jax.ShapeDtypeStruct((N, 1), jnp.float32),
                   jax.ShapeDtypeStruct((N, 1), jnp.float32)],
    )(p0, p1)


def _loop_step(p0, p1, u, xin, xsum, dinv):
    def body(p0_ref, p1_ref, u_ref, x_ref, xs_ref, dv_ref,
             xo_ref, uo_ref, xso_ref):
        xin_v = x_ref[...]
        ps = (p0_ref[...] + p1_ref[...] + u_ref[...])[:, :64]
        xo = dv_ref[...] * ps
        num = jnp.sum(xo * xin_v, axis=1, keepdims=True)
        na = jnp.maximum(
            jnp.sqrt(jnp.sum(xo * xo, axis=1, keepdims=True)), 1e-8)
        nb = jnp.maximum(
            jnp.sqrt(jnp.sum(xin_v * xin_v, axis=1, keepdims=True)), 1e-8)
        w = num / (na * nb)
        xnew = jax.nn.relu(0.5 * (w * xo) + xin_v)
        xo_ref[...] = xnew
        uo_ref[...] = _pad128(dv_ref[...] * xnew)
        xso_ref[...] = xs_ref[...] + xnew

    return pl.pallas_call(
        body,
        grid=(GRID,),
        in_specs=[_row_spec(D), _row_spec(D), _row_spec(D), _row_spec(64),
                  _row_spec(64), _row_spec(1)],
        out_specs=[_row_spec(64), _row_spec(D), _row_spec(64)],
        out_shape=[jax.ShapeDtypeStruct((N, 64), jnp.float32),
                   jax.ShapeDtypeStruct((N, D), jnp.float32),
                   jax.ShapeDtypeStruct((N, 64), jnp.float32)],
    )(p0, p1, u, xin, xsum, dinv)


def _final(t0, t1, g2p, dinv2, b_c2, z_en):
    def body(t0_ref, t1_ref, g_ref, dv_ref, b_ref, z_ref,
             x2_ref, o_ref, q_ref):
        ps = (t0_ref[...] + t1_ref[...] + g_ref[...])[:, :16]
        x2 = jnp.tanh(dv_ref[...] * ps + b_ref[...])
        x2_ref[...] = x2
        o_ref[...] = jax.nn.softmax(x2, axis=1)
        q_ref[...] = jax.nn.softmax(z_ref[...], axis=1)

    return pl.pallas_call(
        body,
        grid=(GRID,),
        in_specs=[_row_spec(D), _row_spec(D), _row_spec(D), _row_spec(1),
                  _full_spec(1, 16), _row_spec(16)],
        out_specs=[_row_spec(16)] * 3,
        out_shape=[jax.ShapeDtypeStruct((N, 16), jnp.float32)] * 3,
    )(t0, t1, g2p, dinv2, b_c2, z_en)


def _identity(v):
    return v


def kernel(x, edge_index, edge_weight, W_enc1, b_enc1, W_enc2, b_enc2, W_enc3,
           b_enc3, W_z, b_z, W_dec1, b_dec1, W_dec2, b_dec2, W_dec3, b_dec3,
           W_xde, b_xde, W_c1, b_c1, W_c2, b_c2):
    src = edge_index[0].astype(jnp.int32).reshape(NW, SUP, SUBC, K)
    dst = edge_index[1].astype(jnp.int32).reshape(NW, SUP, SUBC, K)
    wres = edge_weight.astype(jnp.float32).reshape(NW, SUP, SUBC, K)
    relu = jax.nn.relu

    degp = _deg(dst, wres)
    dinv1, dinv2 = _deg_combine(degp[0], degp[1])

    def spmm1(hp):
        p = _spmm(hp, src, dst)
        return p[0], p[1]

    def spmm2(hp, sw):
        f = {16: _spmm_w16, 64: _spmm_w64}[sw]
        p = f(hp, src, dst, wres)
        return p[0], p[1]

    b1 = b_enc1.reshape(1, -1)
    b2 = b_enc2.reshape(1, -1)
    b3 = b_enc3.reshape(1, -1)
    bz = b_z.reshape(1, -1)
    bd1 = b_dec1.reshape(1, -1)
    bd2 = b_dec2.reshape(1, -1)
    bd3 = b_dec3.reshape(1, -1)
    bxde = b_xde.reshape(1, -1)
    bc1 = b_c1.reshape(1, -1)
    bc2 = b_c2.reshape(1, -1)

    h1p = _mm_scale(x, W_enc1, dinv1)
    p0, p1 = spmm1(h1p)
    _, h2p = _combine(p0, p1, h1p, 64, dinv1, b1, relu, W_next=W_enc2,
                      post_dinv=dinv1)
    p0, p1 = spmm1(h2p)
    _, h3p = _combine(p0, p1, h2p, 64, dinv1, b2, relu, W_next=W_enc3,
                      post_dinv=dinv1)
    p0, p1 = spmm1(h3p)
    _, hzp = _combine(p0, p1, h3p, 64, dinv1, b3, relu, W_next=W_z,
                      post_dinv=dinv1)
    p0, p1 = spmm1(hzp)
    z_en, hd1p = _combine(p0, p1, hzp, 16, dinv1, bz, _identity,
                          W_next=W_dec1, post_dinv=dinv1)
    p0, p1 = spmm1(hd1p)
    _, hd2p = _combine(p0, p1, hd1p, 64, dinv1, bd1, relu, W_next=W_dec2,
                       post_dinv=dinv1)
    p0, p1 = spmm1(hd2p)
    _, hd3p = _combine(p0, p1, hd2p, 64, dinv1, bd2, relu, W_next=W_dec3,
                       post_dinv=dinv1)
    p0, p1 = spmm1(hd3p)
    _, hxp = _combine(p0, p1, hd3p, 64, dinv1, bd3, relu, W_next=W_xde,
                      post_dinv=dinv1)
    p0, p1 = spmm1(hxp)
    x_de = _combine(p0, p1, hxp, 128, dinv1, bxde, _identity)

    g1p = _mm_scale(x, W_c1, dinv2)
    t0, t1 = spmm2(g1p, 64)
    xin, u = _combine(t0, t1, g1p, 64, dinv2, bc1, jax.nn.sigmoid,
                      post_dinv=dinv1)
    xsum = jnp.zeros((N, 64), jnp.float32)
    for _ in range(3):
        p0, p1 = spmm1(u)
        xin, u, xsum = _loop_step(p0, p1, u, xin, xsum, dinv1)
    g2p = _mm_scale(xsum, W_c2, dinv2, scale=1.0 / 3.0)
    t0, t1 = spmm2(g2p, 16)
    x_out2, o, q = _final(t0, t1, g2p, dinv2, bc2, z_en)
    return (x_out2, x_de, z_en, q, o)


# speedup vs baseline: 13.7595x; 1.0585x over previous
import functools

import jax
import jax.numpy as jnp
from jax import lax
from jax.experimental import pallas as pl
from jax.experimental.pallas import tpu as pltpu
from jax.experimental.pallas import tpu_sc as plsc

N = 10000
E = 320000
NCORE = 2
NSUB = 16
NW = NCORE * NSUB
EPT = E // NW
K = 80
KU = 125
SUBC = 5
SUP = EPT // (SUBC * K)
SUPU = EPT // (SUBC * KU)
RPT = 624
SCP = 24
TAIL = N - NSUB * RPT
TAIL_BASE = NSUB * RPT
D = 128

_MESH = plsc.VectorSubcoreMesh(core_axis_name="c", subcore_axis_name="s")


def _make_spmm(weighted, sw=0):
    k = K if weighted else KU
    sup = SUP if weighted else SUPU
    scratch = [
        pltpu.VMEM_SHARED((N, D), jnp.float32),
        pltpu.VMEM((SUBC, k), jnp.int32),
        pltpu.VMEM((SUBC, k), jnp.int32),
        pltpu.VMEM((k, D), jnp.float32),
        pltpu.VMEM((k, D), jnp.float32),
        pltpu.VMEM((SCP, D), jnp.float32),
        pltpu.SemaphoreType.DMA,
        pltpu.SemaphoreType.DMA,
    ]
    if weighted:
        scratch.append(pltpu.VMEM((SUBC, k), jnp.float32))

    @functools.partial(
        pl.kernel,
        mesh=_MESH,
        out_type=jax.ShapeDtypeStruct((NCORE, N, D), jnp.float32),
        scratch_types=scratch,
    )
    def spmm(*refs):
        if weighted:
            (h, srcr, dstr, wr, out, acc, isrc, idst, rows0, rows1, zbuf,
             sem0, sem1, wbuf) = refs
        else:
            (h, srcr, dstr, out, acc, isrc, idst, rows0, rows1, zbuf,
             sem0, sem1) = refs
        bufs = (rows0, rows1)
        sems = (sem0, sem1)
        c = lax.axis_index("c")
        s = lax.axis_index("s")
        wid = s * NCORE + c
        z16 = jnp.zeros((16,), jnp.float32)

        def zrow(r, carry):
            for j in range(D // 16):
                zbuf[r, pl.ds(16 * j, 16)] = z16
            return carry

        lax.fori_loop(0, SCP, zrow, 0)
        base = s * RPT

        def zcp(t, carry):
            off = pl.multiple_of(base + t * SCP, 8)
            pltpu.sync_copy(zbuf, acc.at[pl.ds(off, SCP)])
            return carry

        lax.fori_loop(0, RPT // SCP, zcp, 0)

        @pl.when(s == NSUB - 1)
        def _():
            pltpu.sync_copy(zbuf.at[pl.ds(0, TAIL)],
                            acc.at[pl.ds(TAIL_BASE, TAIL)])

        plsc.subcore_barrier()

        def outer(t, carry):
            pltpu.sync_copy(srcr.at[wid, t], isrc)
            pltpu.sync_copy(dstr.at[wid, t], idst)
            if weighted:
                pltpu.sync_copy(wr.at[wid, t], wbuf)
            handles = [None] * SUBC
            handles[0] = pltpu.async_copy(h.at[isrc.at[0]], bufs[0], sems[0])
            for j in range(SUBC):
                if j + 1 < SUBC:
                    handles[j + 1] = pltpu.async_copy(
                        h.at[isrc.at[j + 1]], bufs[(j + 1) % 2],
                        sems[(j + 1) % 2])
                handles[j].wait()
                rows = bufs[j % 2]
                if weighted:
                    def scale(g, carry2):
                        st = pl.multiple_of(g * 16, 8)
                        w16 = wbuf[j, pl.ds(st, 16)]
                        for b in range(16):
                            wv = w16[b]
                            r = g * 16 + b
                            for q in range(sw // 16):
                                sl = pl.ds(16 * q, 16)
                                rows[r, sl] = rows[r, sl] * wv
                        return carry2

                    lax.fori_loop(0, K // 16, scale, 0)
                pltpu.sync_copy(rows, acc.at[idst.at[j]], add=True)
            return carry

        lax.fori_loop(0, sup, outer, 0)
        plsc.subcore_barrier()

        def wcp(t, carry):
            off = pl.multiple_of(base + t * SCP, 8)
            pltpu.sync_copy(acc.at[pl.ds(off, SCP)],
                            out.at[c, pl.ds(off, SCP)])
            return carry

        lax.fori_loop(0, RPT // SCP, wcp, 0)

        @pl.when(s == NSUB - 1)
        def _():
            pltpu.sync_copy(acc.at[pl.ds(TAIL_BASE, TAIL)],
                            out.at[c, pl.ds(TAIL_BASE, TAIL)])

    return spmm


def _make_deg():
    @functools.partial(
        pl.kernel,
        mesh=_MESH,
        out_type=jax.ShapeDtypeStruct((NCORE, N, D), jnp.float32),
        scratch_types=[
            pltpu.VMEM_SHARED((N, D), jnp.float32),
            pltpu.VMEM((SUBC, K), jnp.int32),
            pltpu.VMEM((SUBC, K), jnp.float32),
            pltpu.VMEM((K, D), jnp.float32),
            pltpu.VMEM((SCP, D), jnp.float32),
        ],
    )
    def deg(dstr, wr, out, acc, idst, wbuf, rows, zbuf):
        c = lax.axis_index("c")
        s = lax.axis_index("s")
        wid = s * NCORE + c
        z16 = jnp.zeros((16,), jnp.float32)
        one16 = jnp.ones((16,), jnp.float32)

        def zrow(r, carry):
            for j in range(D // 16):
                zbuf[r, pl.ds(16 * j, 16)] = z16
            return carry

        lax.fori_loop(0, SCP, zrow, 0)

        def orow(r, carry):
            rows[r, pl.ds(0, 16)] = one16
            for j in range(2, D // 16):
                rows[r, pl.ds(16 * j, 16)] = z16
            return carry

        lax.fori_loop(0, K, orow, 0)
        base = s * RPT

        def zcp(t, carry):
            off = pl.multiple_of(base + t * SCP, 8)
            pltpu.sync_copy(zbuf, acc.at[pl.ds(off, SCP)])
            return carry

        lax.fori_loop(0, RPT // SCP, zcp, 0)

        @pl.when(s == NSUB - 1)
        def _():
            pltpu.sync_copy(zbuf.at[pl.ds(0, TAIL)],
                            acc.at[pl.ds(TAIL_BASE, TAIL)])

        plsc.subcore_barrier()

        def outer(t, carry):
            pltpu.sync_copy(dstr.at[wid, t], idst)
            pltpu.sync_copy(wr.at[wid, t], wbuf)
            for j in range(SUBC):
                def fill(g, carry2):
                    st = pl.multiple_of(g * 16, 8)
                    w16 = wbuf[j, pl.ds(st, 16)]
                    for b in range(16):
                        rows[g * 16 + b, pl.ds(16, 16)] = one16 * w16[b]
                    return carry2

                lax.fori_loop(0, K // 16, fill, 0)
                pltpu.sync_copy(rows, acc.at[idst.at[j]], add=True)
            return carry

        lax.fori_loop(0, SUP, outer, 0)
        plsc.subcore_barrier()

        def wcp(t, carry):
            off = pl.multiple_of(base + t * SCP, 8)
            pltpu.sync_copy(acc.at[pl.ds(off, SCP)],
                            out.at[c, pl.ds(off, SCP)])
            return carry

        lax.fori_loop(0, RPT // SCP, wcp, 0)

        @pl.when(s == NSUB - 1)
        def _():
            pltpu.sync_copy(acc.at[pl.ds(TAIL_BASE, TAIL)],
                            out.at[c, pl.ds(TAIL_BASE, TAIL)])

    return deg


_spmm = _make_spmm(False)
_spmm_w64 = _make_spmm(True, 64)
_spmm_w16 = _make_spmm(True, 16)
_deg = _make_deg()


BR = 1000
GRID = N // BR


def _row_spec(d):
    return pl.BlockSpec((BR, d), lambda i: (i, 0))


def _full_spec(a, b):
    return pl.BlockSpec((a, b), lambda i: (0, 0))


def _pad128(a):
    fo = a.shape[1]
    if fo == D:
        return a
    return jnp.concatenate(
        [a, jnp.zeros((a.shape[0], D - fo), jnp.float32)], axis=1)


def _mm_scale(x, W, dinv, scale=1.0):
    fi, fo = W.shape

    def body(x_ref, w_ref, dv_ref, o_ref):
        acc = jnp.dot(x_ref[...], w_ref[...],
                      preferred_element_type=jnp.float32)
        o_ref[...] = _pad128((dv_ref[...] * scale) * acc)

    return pl.pallas_call(
        body,
        grid=(GRID,),
        in_specs=[_row_spec(fi), _full_spec(fi, fo), _row_spec(1)],
        out_specs=_row_spec(D),
        out_shape=jax.ShapeDtypeStruct((N, D), jnp.float32),
    )(x, W, dinv)


def _combine(p0, p1, hp, d, dinv, b, act, W_next=None, post_dinv=None):
    if W_next is not None:
        def body(p0_ref, p1_ref, hp_ref, dv_ref, b_ref, w_ref, pdv_ref,
                 a_ref, h_ref):
            ps = (p0_ref[...] + p1_ref[...] + hp_ref[...])[:, :d]
            a = act(dv_ref[...] * ps + b_ref[...])
            a_ref[...] = a
            h_ref[...] = _pad128(pdv_ref[...] * jnp.dot(
                a, w_ref[...], preferred_element_type=jnp.float32))

        fo = W_next.shape[1]
        return pl.pallas_call(
            body,
            grid=(GRID,),
            in_specs=[_row_spec(D), _row_spec(D), _row_spec(D), _row_spec(1),
                      _full_spec(1, d), _full_spec(d, fo), _row_spec(1)],
            out_specs=[_row_spec(d), _row_spec(D)],
            out_shape=[jax.ShapeDtypeStruct((N, d), jnp.float32),
                       jax.ShapeDtypeStruct((N, D), jnp.float32)],
        )(p0, p1, hp, dinv, b, W_next, post_dinv)

    if post_dinv is not None:
        def body(p0_ref, p1_ref, hp_ref, dv_ref, b_ref, pdv_ref,
                 a_ref, u_ref):
            ps = (p0_ref[...] + p1_ref[...] + hp_ref[...])[:, :d]
            a = act(dv_ref[...] * ps + b_ref[...])
            a_ref[...] = a
            u_ref[...] = _pad128(pdv_ref[...] * a)

        return pl.pallas_call(
            body,
            grid=(GRID,),
            in_specs=[_row_spec(D), _row_spec(D), _row_spec(D), _row_spec(1),
                      _full_spec(1, d), _row_spec(1)],
            out_specs=[_row_spec(d), _row_spec(D)],
            out_shape=[jax.ShapeDtypeStruct((N, d), jnp.float32),
                       jax.ShapeDtypeStruct((N, D), jnp.float32)],
        )(p0, p1, hp, dinv, b, post_dinv)

    def body(p0_ref, p1_ref, hp_ref, dv_ref, b_ref, a_ref):
        ps = (p0_ref[...] + p1_ref[...] + hp_ref[...])[:, :d]
        a_ref[...] = act(dv_ref[...] * ps + b_ref[...])

    return pl.pallas_call(
        body,
        grid=(GRID,),
        in_specs=[_row_spec(D), _row_spec(D), _row_spec(D), _row_spec(1),
                  _full_spec(1, d)],
        out_specs=_row_spec(d),
        out_shape=jax.ShapeDtypeStruct((N, d), jnp.float32),
    )(p0, p1, hp, dinv, b)


def _deg_combine(p0, p1):
    def body(p0_ref, p1_ref, d1_ref, d2_ref):
        ps = p0_ref[...] + p1_ref[...]
        deg1 = ps[:, 0:1] + 1.0
        deg2 = ps[:, 16:17] + 1.0
        d1_ref[...] = jnp.where(
            deg1 > 0, lax.rsqrt(jnp.maximum(deg1, 1e-12)), 0.0)
        d2_ref[...] = jnp.where(
            deg2 > 0, lax.rsqrt(jnp.maximum(deg2, 1e-12)), 0.0)

    return pl.pallas_call(
        body,
        grid=(GRID,),
        in_specs=[_row_spec(D), _row_spec(D)],
        out_specs=[_row_spec(1), _row_spec(1)],
        out_shape=[jax.ShapeDtypeStruct((N, 1), jnp.float32),
                   jax.ShapeDtypeStruct((N, 1), jnp.float32)],
    )(p0, p1)


def _loop_step(p0, p1, u, xin, xsum, dinv):
    def body(p0_ref, p1_ref, u_ref, x_ref, xs_ref, dv_ref,
             xo_ref, uo_ref, xso_ref):
        xin_v = x_ref[...]
        ps = (p0_ref[...] + p1_ref[...] + u_ref[...])[:, :64]
        xo = dv_ref[...] * ps
        num = jnp.sum(xo * xin_v, axis=1, keepdims=True)
        na = jnp.maximum(
            jnp.sqrt(jnp.sum(xo * xo, axis=1, keepdims=True)), 1e-8)
        nb = jnp.maximum(
            jnp.sqrt(jnp.sum(xin_v * xin_v, axis=1, keepdims=True)), 1e-8)
        w = num / (na * nb)
        xnew = jax.nn.relu(0.5 * (w * xo) + xin_v)
        xo_ref[...] = xnew
        uo_ref[...] = _pad128(dv_ref[...] * xnew)
        xso_ref[...] = xs_ref[...] + xnew

    return pl.pallas_call(
        body,
        grid=(GRID,),
        in_specs=[_row_spec(D), _row_spec(D), _row_spec(D), _row_spec(64),
                  _row_spec(64), _row_spec(1)],
        out_specs=[_row_spec(64), _row_spec(D), _row_spec(64)],
        out_shape=[jax.ShapeDtypeStruct((N, 64), jnp.float32),
                   jax.ShapeDtypeStruct((N, D), jnp.float32),
                   jax.ShapeDtypeStruct((N, 64), jnp.float32)],
    )(p0, p1, u, xin, xsum, dinv)


def _final(t0, t1, g2p, dinv2, b_c2, z_en):
    def body(t0_ref, t1_ref, g_ref, dv_ref, b_ref, z_ref,
             x2_ref, o_ref, q_ref):
        ps = (t0_ref[...] + t1_ref[...] + g_ref[...])[:, :16]
        x2 = jnp.tanh(dv_ref[...] * ps + b_ref[...])
        x2_ref[...] = x2
        o_ref[...] = jax.nn.softmax(x2, axis=1)
        q_ref[...] = jax.nn.softmax(z_ref[...], axis=1)

    return pl.pallas_call(
        body,
        grid=(GRID,),
        in_specs=[_row_spec(D), _row_spec(D), _row_spec(D), _row_spec(1),
                  _full_spec(1, 16), _row_spec(16)],
        out_specs=[_row_spec(16)] * 3,
        out_shape=[jax.ShapeDtypeStruct((N, 16), jnp.float32)] * 3,
    )(t0, t1, g2p, dinv2, b_c2, z_en)


def _identity(v):
    return v


def kernel(x, edge_index, edge_weight, W_enc1, b_enc1, W_enc2, b_enc2, W_enc3,
           b_enc3, W_z, b_z, W_dec1, b_dec1, W_dec2, b_dec2, W_dec3, b_dec3,
           W_xde, b_xde, W_c1, b_c1, W_c2, b_c2):
    srci = edge_index[0].astype(jnp.int32)
    dsti = edge_index[1].astype(jnp.int32)
    src = srci.reshape(NW, SUP, SUBC, K)
    dst = dsti.reshape(NW, SUP, SUBC, K)
    srcu = srci.reshape(NW, SUPU, SUBC, KU)
    dstu = dsti.reshape(NW, SUPU, SUBC, KU)
    wres = edge_weight.astype(jnp.float32).reshape(NW, SUP, SUBC, K)
    relu = jax.nn.relu

    degp = _deg(dst, wres)
    dinv1, dinv2 = _deg_combine(degp[0], degp[1])

    def spmm1(hp):
        p = _spmm(hp, srcu, dstu)
        return p[0], p[1]

    def spmm2(hp, sw):
        f = {16: _spmm_w16, 64: _spmm_w64}[sw]
        p = f(hp, src, dst, wres)
        return p[0], p[1]

    b1 = b_enc1.reshape(1, -1)
    b2 = b_enc2.reshape(1, -1)
    b3 = b_enc3.reshape(1, -1)
    bz = b_z.reshape(1, -1)
    bd1 = b_dec1.reshape(1, -1)
    bd2 = b_dec2.reshape(1, -1)
    bd3 = b_dec3.reshape(1, -1)
    bxde = b_xde.reshape(1, -1)
    bc1 = b_c1.reshape(1, -1)
    bc2 = b_c2.reshape(1, -1)

    h1p = _mm_scale(x, W_enc1, dinv1)
    p0, p1 = spmm1(h1p)
    _, h2p = _combine(p0, p1, h1p, 64, dinv1, b1, relu, W_next=W_enc2,
                      post_dinv=dinv1)
    p0, p1 = spmm1(h2p)
    _, h3p = _combine(p0, p1, h2p, 64, dinv1, b2, relu, W_next=W_enc3,
                      post_dinv=dinv1)
    p0, p1 = spmm1(h3p)
    _, hzp = _combine(p0, p1, h3p, 64, dinv1, b3, relu, W_next=W_z,
                      post_dinv=dinv1)
    p0, p1 = spmm1(hzp)
    z_en, hd1p = _combine(p0, p1, hzp, 16, dinv1, bz, _identity,
                          W_next=W_dec1, post_dinv=dinv1)
    p0, p1 = spmm1(hd1p)
    _, hd2p = _combine(p0, p1, hd1p, 64, dinv1, bd1, relu, W_next=W_dec2,
                       post_dinv=dinv1)
    p0, p1 = spmm1(hd2p)
    _, hd3p = _combine(p0, p1, hd2p, 64, dinv1, bd2, relu, W_next=W_dec3,
                       post_dinv=dinv1)
    p0, p1 = spmm1(hd3p)
    _, hxp = _combine(p0, p1, hd3p, 64, dinv1, bd3, relu, W_next=W_xde,
                      post_dinv=dinv1)
    p0, p1 = spmm1(hxp)
    x_de = _combine(p0, p1, hxp, 128, dinv1, bxde, _identity)

    g1p = _mm_scale(x, W_c1, dinv2)
    t0, t1 = spmm2(g1p, 64)
    xin, u = _combine(t0, t1, g1p, 64, dinv2, bc1, jax.nn.sigmoid,
                      post_dinv=dinv1)
    xsum = jnp.zeros((N, 64), jnp.float32)
    for _ in range(3):
        p0, p1 = spmm1(u)
        xin, u, xsum = _loop_step(p0, p1, u, xin, xsum, dinv1)
    g2p = _mm_scale(xsum, W_c2, dinv2, scale=1.0 / 3.0)
    t0, t1 = spmm2(g2p, 16)
    x_out2, o, q = _final(t0, t1, g2p, dinv2, bc2, z_en)
    return (x_out2, x_de, z_en, q, o)


# speedup vs baseline: 15.4804x; 1.1251x over previous
import functools

import jax
import jax.numpy as jnp
from jax import lax
from jax.experimental import pallas as pl
from jax.experimental.pallas import tpu as pltpu
from jax.experimental.pallas import tpu_sc as plsc

N = 10000
E = 320000
NCORE = 2
NSUB = 16
NW = NCORE * NSUB
EPT = E // NW
K = 80
KU = 125
SUBC = 5
SUP = EPT // (SUBC * K)
SUPU = EPT // (SUBC * KU)
RPT = 624
SCP = 24
TAIL = N - NSUB * RPT
TAIL_BASE = NSUB * RPT
D = 128

_MESH = plsc.VectorSubcoreMesh(core_axis_name="c", subcore_axis_name="s")


def _make_spmm(weighted, sw=0):
    k = K if weighted else KU
    sup = SUP if weighted else SUPU
    scratch = [
        pltpu.VMEM_SHARED((N, D), jnp.float32),
        pltpu.VMEM((2, SUBC, k), jnp.int32),
        pltpu.VMEM((2, SUBC, k), jnp.int32),
        pltpu.VMEM((k, D), jnp.float32),
        pltpu.VMEM((k, D), jnp.float32),
        pltpu.VMEM((SCP, D), jnp.float32),
        pltpu.SemaphoreType.DMA,
        pltpu.SemaphoreType.DMA,
        pltpu.SemaphoreType.DMA,
    ]
    if weighted:
        scratch.append(pltpu.VMEM((2, SUBC, k), jnp.float32))

    @functools.partial(
        pl.kernel,
        mesh=_MESH,
        out_type=jax.ShapeDtypeStruct((NCORE, N, D), jnp.float32),
        scratch_types=scratch,
    )
    def spmm(*refs):
        if weighted:
            (h, srcr, dstr, wr, out, acc, isrc, idst, rows0, rows1, zbuf,
             sem0, sem1, isem, wbuf) = refs
        else:
            (h, srcr, dstr, out, acc, isrc, idst, rows0, rows1, zbuf,
             sem0, sem1, isem) = refs
        bufs = (rows0, rows1)
        sems = (sem0, sem1)
        c = lax.axis_index("c")
        s = lax.axis_index("s")
        wid = s * NCORE + c
        z16 = jnp.zeros((16,), jnp.float32)

        def zrow(r, carry):
            for j in range(D // 16):
                zbuf[r, pl.ds(16 * j, 16)] = z16
            return carry

        lax.fori_loop(0, SCP, zrow, 0)
        base = s * RPT

        def zcp(t, carry):
            off = pl.multiple_of(base + t * SCP, 8)
            pltpu.sync_copy(zbuf, acc.at[pl.ds(off, SCP)])
            return carry

        lax.fori_loop(0, RPT // SCP, zcp, 0)

        @pl.when(s == NSUB - 1)
        def _():
            pltpu.sync_copy(zbuf.at[pl.ds(0, TAIL)],
                            acc.at[pl.ds(TAIL_BASE, TAIL)])

        pltpu.async_copy(srcr.at[wid, 0], isrc.at[0], isem)
        pltpu.async_copy(dstr.at[wid, 0], idst.at[0], isem)
        if weighted:
            pltpu.async_copy(wr.at[wid, 0], wbuf.at[0], isem)
        plsc.subcore_barrier()

        def outer(t, carry):
            par = t % 2
            pltpu.make_async_copy(srcr.at[wid, t], isrc.at[par], isem).wait()
            pltpu.make_async_copy(dstr.at[wid, t], idst.at[par], isem).wait()
            if weighted:
                pltpu.make_async_copy(wr.at[wid, t], wbuf.at[par], isem).wait()

            @pl.when(t + 1 < sup)
            def _():
                pltpu.async_copy(srcr.at[wid, t + 1], isrc.at[1 - par], isem)
                pltpu.async_copy(dstr.at[wid, t + 1], idst.at[1 - par], isem)
                if weighted:
                    pltpu.async_copy(wr.at[wid, t + 1], wbuf.at[1 - par], isem)

            isr = isrc.at[par]
            ids = idst.at[par]
            handles = [None] * SUBC
            handles[0] = pltpu.async_copy(h.at[isr.at[0]], bufs[0], sems[0])
            for j in range(SUBC):
                if j + 1 < SUBC:
                    handles[j + 1] = pltpu.async_copy(
                        h.at[isr.at[j + 1]], bufs[(j + 1) % 2],
                        sems[(j + 1) % 2])
                handles[j].wait()
                rows = bufs[j % 2]
                if weighted:
                    def scale(g, carry2):
                        st = pl.multiple_of(g * 16, 8)
                        w16 = wbuf[par, j, pl.ds(st, 16)]
                        for b in range(16):
                            wv = w16[b]
                            r = g * 16 + b
                            for q in range(sw // 16):
                                sl = pl.ds(16 * q, 16)
                                rows[r, sl] = rows[r, sl] * wv
                        return carry2

                    lax.fori_loop(0, K // 16, scale, 0)
                pltpu.sync_copy(rows, acc.at[ids.at[j]], add=True)
            return carry

        lax.fori_loop(0, sup, outer, 0)
        plsc.subcore_barrier()

        def wcp(t, carry):
            off = pl.multiple_of(base + t * SCP, 8)
            pltpu.sync_copy(acc.at[pl.ds(off, SCP)],
                            out.at[c, pl.ds(off, SCP)])
            return carry

        lax.fori_loop(0, RPT // SCP, wcp, 0)

        @pl.when(s == NSUB - 1)
        def _():
            pltpu.sync_copy(acc.at[pl.ds(TAIL_BASE, TAIL)],
                            out.at[c, pl.ds(TAIL_BASE, TAIL)])

    return spmm


def _make_deg():
    @functools.partial(
        pl.kernel,
        mesh=_MESH,
        out_type=jax.ShapeDtypeStruct((NCORE, N, D), jnp.float32),
        scratch_types=[
            pltpu.VMEM_SHARED((N, D), jnp.float32),
            pltpu.VMEM((SUBC, K), jnp.int32),
            pltpu.VMEM((SUBC, K), jnp.float32),
            pltpu.VMEM((K, D), jnp.float32),
            pltpu.VMEM((SCP, D), jnp.float32),
        ],
    )
    def deg(dstr, wr, out, acc, idst, wbuf, rows, zbuf):
        c = lax.axis_index("c")
        s = lax.axis_index("s")
        wid = s * NCORE + c
        z16 = jnp.zeros((16,), jnp.float32)
        one16 = jnp.ones((16,), jnp.float32)

        def zrow(r, carry):
            for j in range(D // 16):
                zbuf[r, pl.ds(16 * j, 16)] = z16
            return carry

        lax.fori_loop(0, SCP, zrow, 0)

        def orow(r, carry):
            rows[r, pl.ds(0, 16)] = one16
            for j in range(2, D // 16):
                rows[r, pl.ds(16 * j, 16)] = z16
            return carry

        lax.fori_loop(0, K, orow, 0)
        base = s * RPT

        def zcp(t, carry):
            off = pl.multiple_of(base + t * SCP, 8)
            pltpu.sync_copy(zbuf, acc.at[pl.ds(off, SCP)])
            return carry

        lax.fori_loop(0, RPT // SCP, zcp, 0)

        @pl.when(s == NSUB - 1)
        def _():
            pltpu.sync_copy(zbuf.at[pl.ds(0, TAIL)],
                            acc.at[pl.ds(TAIL_BASE, TAIL)])

        plsc.subcore_barrier()

        def outer(t, carry):
            pltpu.sync_copy(dstr.at[wid, t], idst)
            pltpu.sync_copy(wr.at[wid, t], wbuf)
            for j in range(SUBC):
                def fill(g, carry2):
                    st = pl.multiple_of(g * 16, 8)
                    w16 = wbuf[j, pl.ds(st, 16)]
                    for b in range(16):
                        rows[g * 16 + b, pl.ds(16, 16)] = one16 * w16[b]
                    return carry2

                lax.fori_loop(0, K // 16, fill, 0)
                pltpu.sync_copy(rows, acc.at[idst.at[j]], add=True)
            return carry

        lax.fori_loop(0, SUP, outer, 0)
        plsc.subcore_barrier()

        def wcp(t, carry):
            off = pl.multiple_of(base + t * SCP, 8)
            pltpu.sync_copy(acc.at[pl.ds(off, SCP)],
                            out.at[c, pl.ds(off, SCP)])
            return carry

        lax.fori_loop(0, RPT // SCP, wcp, 0)

        @pl.when(s == NSUB - 1)
        def _():
            pltpu.sync_copy(acc.at[pl.ds(TAIL_BASE, TAIL)],
                            out.at[c, pl.ds(TAIL_BASE, TAIL)])

    return deg


_spmm = _make_spmm(False)
_spmm_w64 = _make_spmm(True, 64)
_spmm_w16 = _make_spmm(True, 16)
_deg = _make_deg()


BR = 1000
GRID = N // BR


def _row_spec(d):
    return pl.BlockSpec((BR, d), lambda i: (i, 0))


def _full_spec(a, b):
    return pl.BlockSpec((a, b), lambda i: (0, 0))


def _pad128(a):
    fo = a.shape[1]
    if fo == D:
        return a
    return jnp.concatenate(
        [a, jnp.zeros((a.shape[0], D - fo), jnp.float32)], axis=1)


def _mm_scale(x, W, dinv, scale=1.0):
    fi, fo = W.shape

    def body(x_ref, w_ref, dv_ref, o_ref):
        acc = jnp.dot(x_ref[...], w_ref[...],
                      preferred_element_type=jnp.float32)
        o_ref[...] = _pad128((dv_ref[...] * scale) * acc)

    return pl.pallas_call(
        body,
        grid=(GRID,),
        in_specs=[_row_spec(fi), _full_spec(fi, fo), _row_spec(1)],
        out_specs=_row_spec(D),
        out_shape=jax.ShapeDtypeStruct((N, D), jnp.float32),
    )(x, W, dinv)


def _combine(p0, p1, hp, d, dinv, b, act, W_next=None, post_dinv=None):
    if W_next is not None:
        def body(p0_ref, p1_ref, hp_ref, dv_ref, b_ref, w_ref, pdv_ref,
                 a_ref, h_ref):
            ps = (p0_ref[...] + p1_ref[...] + hp_ref[...])[:, :d]
            a = act(dv_ref[...] * ps + b_ref[...])
            a_ref[...] = a
            h_ref[...] = _pad128(pdv_ref[...] * jnp.dot(
                a, w_ref[...], preferred_element_type=jnp.float32))

        fo = W_next.shape[1]
        return pl.pallas_call(
            body,
            grid=(GRID,),
            in_specs=[_row_spec(D), _row_spec(D), _row_spec(D), _row_spec(1),
                      _full_spec(1, d), _full_spec(d, fo), _row_spec(1)],
            out_specs=[_row_spec(d), _row_spec(D)],
            out_shape=[jax.ShapeDtypeStruct((N, d), jnp.float32),
                       jax.ShapeDtypeStruct((N, D), jnp.float32)],
        )(p0, p1, hp, dinv, b, W_next, post_dinv)

    if post_dinv is not None:
        def body(p0_ref, p1_ref, hp_ref, dv_ref, b_ref, pdv_ref,
                 a_ref, u_ref):
            ps = (p0_ref[...] + p1_ref[...] + hp_ref[...])[:, :d]
            a = act(dv_ref[...] * ps + b_ref[...])
            a_ref[...] = a
            u_ref[...] = _pad128(pdv_ref[...] * a)

        return pl.pallas_call(
            body,
            grid=(GRID,),
            in_specs=[_row_spec(D), _row_spec(D), _row_spec(D), _row_spec(1),
                      _full_spec(1, d), _row_spec(1)],
            out_specs=[_row_spec(d), _row_spec(D)],
            out_shape=[jax.ShapeDtypeStruct((N, d), jnp.float32),
                       jax.ShapeDtypeStruct((N, D), jnp.float32)],
        )(p0, p1, hp, dinv, b, post_dinv)

    def body(p0_ref, p1_ref, hp_ref, dv_ref, b_ref, a_ref):
        ps = (p0_ref[...] + p1_ref[...] + hp_ref[...])[:, :d]
        a_ref[...] = act(dv_ref[...] * ps + b_ref[...])

    return pl.pallas_call(
        body,
        grid=(GRID,),
        in_specs=[_row_spec(D), _row_spec(D), _row_spec(D), _row_spec(1),
                  _full_spec(1, d)],
        out_specs=_row_spec(d),
        out_shape=jax.ShapeDtypeStruct((N, d), jnp.float32),
    )(p0, p1, hp, dinv, b)


def _deg_combine(p0, p1):
    def body(p0_ref, p1_ref, d1_ref, d2_ref):
        ps = p0_ref[...] + p1_ref[...]
        deg1 = ps[:, 0:1] + 1.0
        deg2 = ps[:, 16:17] + 1.0
        d1_ref[...] = jnp.where(
            deg1 > 0, lax.rsqrt(jnp.maximum(deg1, 1e-12)), 0.0)
        d2_ref[...] = jnp.where(
            deg2 > 0, lax.rsqrt(jnp.maximum(deg2, 1e-12)), 0.0)

    return pl.pallas_call(
        body,
        grid=(GRID,),
        in_specs=[_row_spec(D), _row_spec(D)],
        out_specs=[_row_spec(1), _row_spec(1)],
        out_shape=[jax.ShapeDtypeStruct((N, 1), jnp.float32),
                   jax.ShapeDtypeStruct((N, 1), jnp.float32)],
    )(p0, p1)


def _loop_step(p0, p1, u, xin, xsum, dinv):
    def body(p0_ref, p1_ref, u_ref, x_ref, xs_ref, dv_ref,
             xo_ref, uo_ref, xso_ref):
        xin_v = x_ref[...]
        ps = (p0_ref[...] + p1_ref[...] + u_ref[...])[:, :64]
        xo = dv_ref[...] * ps
        num = jnp.sum(xo * xin_v, axis=1, keepdims=True)
        na = jnp.maximum(
            jnp.sqrt(jnp.sum(xo * xo, axis=1, keepdims=True)), 1e-8)
        nb = jnp.maximum(
            jnp.sqrt(jnp.sum(xin_v * xin_v, axis=1, keepdims=True)), 1e-8)
        w = num / (na * nb)
        xnew = jax.nn.relu(0.5 * (w * xo) + xin_v)
        xo_ref[...] = xnew
        uo_ref[...] = _pad128(dv_ref[...] * xnew)
        xso_ref[...] = xs_ref[...] + xnew

    return pl.pallas_call(
        body,
        grid=(GRID,),
        in_specs=[_row_spec(D), _row_spec(D), _row_spec(D), _row_spec(64),
                  _row_spec(64), _row_spec(1)],
        out_specs=[_row_spec(64), _row_spec(D), _row_spec(64)],
        out_shape=[jax.ShapeDtypeStruct((N, 64), jnp.float32),
                   jax.ShapeDtypeStruct((N, D), jnp.float32),
                   jax.ShapeDtypeStruct((N, 64), jnp.float32)],
    )(p0, p1, u, xin, xsum, dinv)


def _final(t0, t1, g2p, dinv2, b_c2, z_en):
    def body(t0_ref, t1_ref, g_ref, dv_ref, b_ref, z_ref,
             x2_ref, o_ref, q_ref):
        ps = (t0_ref[...] + t1_ref[...] + g_ref[...])[:, :16]
        x2 = jnp.tanh(dv_ref[...] * ps + b_ref[...])
        x2_ref[...] = x2
        o_ref[...] = jax.nn.softmax(x2, axis=1)
        q_ref[...] = jax.nn.softmax(z_ref[...], axis=1)

    return pl.pallas_call(
        body,
        grid=(GRID,),
        in_specs=[_row_spec(D), _row_spec(D), _row_spec(D), _row_spec(1),
                  _full_spec(1, 16), _row_spec(16)],
        out_specs=[_row_spec(16)] * 3,
        out_shape=[jax.ShapeDtypeStruct((N, 16), jnp.float32)] * 3,
    )(t0, t1, g2p, dinv2, b_c2, z_en)


def _identity(v):
    return v


def kernel(x, edge_index, edge_weight, W_enc1, b_enc1, W_enc2, b_enc2, W_enc3,
           b_enc3, W_z, b_z, W_dec1, b_dec1, W_dec2, b_dec2, W_dec3, b_dec3,
           W_xde, b_xde, W_c1, b_c1, W_c2, b_c2):
    srci = edge_index[0].astype(jnp.int32)
    dsti = edge_index[1].astype(jnp.int32)
    src = srci.reshape(NW, SUP, SUBC, K)
    dst = dsti.reshape(NW, SUP, SUBC, K)
    srcu = srci.reshape(NW, SUPU, SUBC, KU)
    dstu = dsti.reshape(NW, SUPU, SUBC, KU)
    wres = edge_weight.astype(jnp.float32).reshape(NW, SUP, SUBC, K)
    relu = jax.nn.relu

    degp = _deg(dst, wres)
    dinv1, dinv2 = _deg_combine(degp[0], degp[1])

    def spmm1(hp):
        p = _spmm(hp, srcu, dstu)
        return p[0], p[1]

    def spmm2(hp, sw):
        f = {16: _spmm_w16, 64: _spmm_w64}[sw]
        p = f(hp, src, dst, wres)
        return p[0], p[1]

    b1 = b_enc1.reshape(1, -1)
    b2 = b_enc2.reshape(1, -1)
    b3 = b_enc3.reshape(1, -1)
    bz = b_z.reshape(1, -1)
    bd1 = b_dec1.reshape(1, -1)
    bd2 = b_dec2.reshape(1, -1)
    bd3 = b_dec3.reshape(1, -1)
    bxde = b_xde.reshape(1, -1)
    bc1 = b_c1.reshape(1, -1)
    bc2 = b_c2.reshape(1, -1)

    h1p = _mm_scale(x, W_enc1, dinv1)
    p0, p1 = spmm1(h1p)
    _, h2p = _combine(p0, p1, h1p, 64, dinv1, b1, relu, W_next=W_enc2,
                      post_dinv=dinv1)
    p0, p1 = spmm1(h2p)
    _, h3p = _combine(p0, p1, h2p, 64, dinv1, b2, relu, W_next=W_enc3,
                      post_dinv=dinv1)
    p0, p1 = spmm1(h3p)
    _, hzp = _combine(p0, p1, h3p, 64, dinv1, b3, relu, W_next=W_z,
                      post_dinv=dinv1)
    p0, p1 = spmm1(hzp)
    z_en, hd1p = _combine(p0, p1, hzp, 16, dinv1, bz, _identity,
                          W_next=W_dec1, post_dinv=dinv1)
    p0, p1 = spmm1(hd1p)
    _, hd2p = _combine(p0, p1, hd1p, 64, dinv1, bd1, relu, W_next=W_dec2,
                       post_dinv=dinv1)
    p0, p1 = spmm1(hd2p)
    _, hd3p = _combine(p0, p1, hd2p, 64, dinv1, bd2, relu, W_next=W_dec3,
                       post_dinv=dinv1)
    p0, p1 = spmm1(hd3p)
    _, hxp = _combine(p0, p1, hd3p, 64, dinv1, bd3, relu, W_next=W_xde,
                      post_dinv=dinv1)
    p0, p1 = spmm1(hxp)
    x_de = _combine(p0, p1, hxp, 128, dinv1, bxde, _identity)

    g1p = _mm_scale(x, W_c1, dinv2)
    t0, t1 = spmm2(g1p, 64)
    xin, u = _combine(t0, t1, g1p, 64, dinv2, bc1, jax.nn.sigmoid,
                      post_dinv=dinv1)
    xsum = jnp.zeros((N, 64), jnp.float32)
    for _ in range(3):
        p0, p1 = spmm1(u)
        xin, u, xsum = _loop_step(p0, p1, u, xin, xsum, dinv1)
    g2p = _mm_scale(xsum, W_c2, dinv2, scale=1.0 / 3.0)
    t0, t1 = spmm2(g2p, 16)
    x_out2, o, q = _final(t0, t1, g2p, dinv2, bc2, z_en)
    return (x_out2, x_de, z_en, q, o)


# speedup vs baseline: 16.5573x; 1.0696x over previous
import functools

import jax
import jax.numpy as jnp
from jax import lax
from jax.experimental import pallas as pl
from jax.experimental.pallas import tpu as pltpu
from jax.experimental.pallas import tpu_sc as plsc

N = 10000
E = 320000
NCORE = 2
NSUB = 16
NW = NCORE * NSUB
EPT = E // NW
K = 80
KU = 125
SUBC = 5
SUP = EPT // (SUBC * K)
SUPU = EPT // (SUBC * KU)
RPT = 624
SCP = 24
TAIL = N - NSUB * RPT
TAIL_BASE = NSUB * RPT
D = 128

_MESH = plsc.VectorSubcoreMesh(core_axis_name="c", subcore_axis_name="s")


def _make_spmm(weighted, sw=0):
    k = K if weighted else KU
    sup = SUP if weighted else SUPU
    scratch = [
        pltpu.VMEM_SHARED((N, D), jnp.float32),
        pltpu.VMEM((2, SUBC, k), jnp.int32),
        pltpu.VMEM((2, SUBC, k), jnp.int32),
        pltpu.VMEM((k, D), jnp.float32),
        pltpu.VMEM((k, D), jnp.float32),
        pltpu.VMEM((SCP, D), jnp.float32),
        pltpu.SemaphoreType.DMA,
        pltpu.SemaphoreType.DMA,
        pltpu.SemaphoreType.DMA,
    ]
    if weighted:
        scratch.append(pltpu.VMEM((2, SUBC, k), jnp.float32))

    @functools.partial(
        pl.kernel,
        mesh=_MESH,
        out_type=jax.ShapeDtypeStruct((NCORE, N, D), jnp.float32),
        scratch_types=scratch,
    )
    def spmm(*refs):
        if weighted:
            (h, srcr, dstr, wr, out, acc, isrc, idst, rows0, rows1, zbuf,
             sem0, sem1, isem, wbuf) = refs
        else:
            (h, srcr, dstr, out, acc, isrc, idst, rows0, rows1, zbuf,
             sem0, sem1, isem) = refs
        bufs = (rows0, rows1)
        sems = (sem0, sem1)
        c = lax.axis_index("c")
        s = lax.axis_index("s")
        wid = s * NCORE + c
        z16 = jnp.zeros((16,), jnp.float32)

        def zrow(r, carry):
            for j in range(D // 16):
                zbuf[r, pl.ds(16 * j, 16)] = z16
            return carry

        lax.fori_loop(0, SCP, zrow, 0)
        base = s * RPT

        def zcp(t, carry):
            off = pl.multiple_of(base + t * SCP, 8)
            pltpu.async_copy(zbuf, acc.at[pl.ds(off, SCP)], sem0)
            return carry

        lax.fori_loop(0, RPT // SCP, zcp, 0)

        @pl.when(s == NSUB - 1)
        def _():
            pltpu.async_copy(zbuf.at[pl.ds(0, TAIL)],
                            acc.at[pl.ds(TAIL_BASE, TAIL)], sem0)

        def zdr(t, carry):
            off = pl.multiple_of(base + t * SCP, 8)
            pltpu.make_async_copy(zbuf, acc.at[pl.ds(off, SCP)], sem0).wait()
            return carry

        lax.fori_loop(0, RPT // SCP, zdr, 0)

        @pl.when(s == NSUB - 1)
        def _():
            pltpu.make_async_copy(zbuf.at[pl.ds(0, TAIL)],
                                  acc.at[pl.ds(TAIL_BASE, TAIL)], sem0).wait()

        pltpu.async_copy(srcr.at[wid, 0], isrc.at[0], isem)
        pltpu.async_copy(dstr.at[wid, 0], idst.at[0], isem)
        if weighted:
            pltpu.async_copy(wr.at[wid, 0], wbuf.at[0], isem)
        plsc.subcore_barrier()

        def outer(t, carry):
            par = t % 2
            pltpu.make_async_copy(srcr.at[wid, t], isrc.at[par], isem).wait()
            pltpu.make_async_copy(dstr.at[wid, t], idst.at[par], isem).wait()
            if weighted:
                pltpu.make_async_copy(wr.at[wid, t], wbuf.at[par], isem).wait()

            @pl.when(t + 1 < sup)
            def _():
                pltpu.async_copy(srcr.at[wid, t + 1], isrc.at[1 - par], isem)
                pltpu.async_copy(dstr.at[wid, t + 1], idst.at[1 - par], isem)
                if weighted:
                    pltpu.async_copy(wr.at[wid, t + 1], wbuf.at[1 - par], isem)

            isr = isrc.at[par]
            ids = idst.at[par]
            handles = [None] * SUBC
            handles[0] = pltpu.async_copy(h.at[isr.at[0]], bufs[0], sems[0])
            for j in range(SUBC):
                if j + 1 < SUBC:
                    handles[j + 1] = pltpu.async_copy(
                        h.at[isr.at[j + 1]], bufs[(j + 1) % 2],
                        sems[(j + 1) % 2])
                handles[j].wait()
                rows = bufs[j % 2]
                if weighted:
                    def scale(g, carry2):
                        st = pl.multiple_of(g * 16, 8)
                        w16 = wbuf[par, j, pl.ds(st, 16)]
                        for b in range(16):
                            wv = w16[b]
                            r = g * 16 + b
                            for q in range(sw // 16):
                                sl = pl.ds(16 * q, 16)
                                rows[r, sl] = rows[r, sl] * wv
                        return carry2

                    lax.fori_loop(0, K // 16, scale, 0)
                pltpu.sync_copy(rows, acc.at[ids.at[j]], add=True)
            return carry

        lax.fori_loop(0, sup, outer, 0)
        plsc.subcore_barrier()

        def wcp(t, carry):
            off = pl.multiple_of(base + t * SCP, 8)
            pltpu.async_copy(acc.at[pl.ds(off, SCP)],
                             out.at[c, pl.ds(off, SCP)], sem0)
            return carry

        lax.fori_loop(0, RPT // SCP, wcp, 0)

        @pl.when(s == NSUB - 1)
        def _():
            pltpu.async_copy(acc.at[pl.ds(TAIL_BASE, TAIL)],
                             out.at[c, pl.ds(TAIL_BASE, TAIL)], sem0)

        def wdr(t, carry):
            off = pl.multiple_of(base + t * SCP, 8)
            pltpu.make_async_copy(acc.at[pl.ds(off, SCP)],
                                  out.at[c, pl.ds(off, SCP)], sem0).wait()
            return carry

        lax.fori_loop(0, RPT // SCP, wdr, 0)

        @pl.when(s == NSUB - 1)
        def _():
            pltpu.make_async_copy(acc.at[pl.ds(TAIL_BASE, TAIL)],
                                  out.at[c, pl.ds(TAIL_BASE, TAIL)],
                                  sem0).wait()

    return spmm


def _make_deg():
    @functools.partial(
        pl.kernel,
        mesh=_MESH,
        out_type=jax.ShapeDtypeStruct((NCORE, N, D), jnp.float32),
        scratch_types=[
            pltpu.VMEM_SHARED((N, D), jnp.float32),
            pltpu.VMEM((SUBC, K), jnp.int32),
            pltpu.VMEM((SUBC, K), jnp.float32),
            pltpu.VMEM((K, D), jnp.float32),
            pltpu.VMEM((SCP, D), jnp.float32),
        ],
    )
    def deg(dstr, wr, out, acc, idst, wbuf, rows, zbuf):
        c = lax.axis_index("c")
        s = lax.axis_index("s")
        wid = s * NCORE + c
        z16 = jnp.zeros((16,), jnp.float32)
        one16 = jnp.ones((16,), jnp.float32)

        def zrow(r, carry):
            for j in range(D // 16):
                zbuf[r, pl.ds(16 * j, 16)] = z16
            return carry

        lax.fori_loop(0, SCP, zrow, 0)

        def orow(r, carry):
            rows[r, pl.ds(0, 16)] = one16
            for j in range(2, D // 16):
                rows[r, pl.ds(16 * j, 16)] = z16
            return carry

        lax.fori_loop(0, K, orow, 0)
        base = s * RPT

        def zcp(t, carry):
            off = pl.multiple_of(base + t * SCP, 8)
            pltpu.sync_copy(zbuf, acc.at[pl.ds(off, SCP)])
            return carry

        lax.fori_loop(0, RPT // SCP, zcp, 0)

        @pl.when(s == NSUB - 1)
        def _():
            pltpu.sync_copy(zbuf.at[pl.ds(0, TAIL)],
                            acc.at[pl.ds(TAIL_BASE, TAIL)])

        plsc.subcore_barrier()

        def outer(t, carry):
            pltpu.sync_copy(dstr.at[wid, t], idst)
            pltpu.sync_copy(wr.at[wid, t], wbuf)
            for j in range(SUBC):
                def fill(g, carry2):
                    st = pl.multiple_of(g * 16, 8)
                    w16 = wbuf[j, pl.ds(st, 16)]
                    for b in range(16):
                        rows[g * 16 + b, pl.ds(16, 16)] = one16 * w16[b]
                    return carry2

                lax.fori_loop(0, K // 16, fill, 0)
                pltpu.sync_copy(rows, acc.at[idst.at[j]], add=True)
            return carry

        lax.fori_loop(0, SUP, outer, 0)
        plsc.subcore_barrier()

        def wcp(t, carry):
            off = pl.multiple_of(base + t * SCP, 8)
            pltpu.sync_copy(acc.at[pl.ds(off, SCP)],
                            out.at[c, pl.ds(off, SCP)])
            return carry

        lax.fori_loop(0, RPT // SCP, wcp, 0)

        @pl.when(s == NSUB - 1)
        def _():
            pltpu.sync_copy(acc.at[pl.ds(TAIL_BASE, TAIL)],
                            out.at[c, pl.ds(TAIL_BASE, TAIL)])

    return deg


_spmm = _make_spmm(False)
_spmm_w64 = _make_spmm(True, 64)
_spmm_w16 = _make_spmm(True, 16)
_deg = _make_deg()


BR = 1000
GRID = N // BR


def _row_spec(d):
    return pl.BlockSpec((BR, d), lambda i: (i, 0))


def _full_spec(a, b):
    return pl.BlockSpec((a, b), lambda i: (0, 0))


def _pad128(a):
    fo = a.shape[1]
    if fo == D:
        return a
    return jnp.concatenate(
        [a, jnp.zeros((a.shape[0], D - fo), jnp.float32)], axis=1)


def _mm_scale(x, W, dinv, scale=1.0):
    fi, fo = W.shape

    def body(x_ref, w_ref, dv_ref, o_ref):
        acc = jnp.dot(x_ref[...], w_ref[...],
                      preferred_element_type=jnp.float32)
        o_ref[...] = _pad128((dv_ref[...] * scale) * acc)

    return pl.pallas_call(
        body,
        grid=(GRID,),
        in_specs=[_row_spec(fi), _full_spec(fi, fo), _row_spec(1)],
        out_specs=_row_spec(D),
        out_shape=jax.ShapeDtypeStruct((N, D), jnp.float32),
    )(x, W, dinv)


def _combine(p0, p1, hp, d, dinv, b, act, W_next=None, post_dinv=None):
    if W_next is not None:
        def body(p0_ref, p1_ref, hp_ref, dv_ref, b_ref, w_ref, pdv_ref,
                 a_ref, h_ref):
            ps = (p0_ref[...] + p1_ref[...] + hp_ref[...])[:, :d]
            a = act(dv_ref[...] * ps + b_ref[...])
            a_ref[...] = a
            h_ref[...] = _pad128(pdv_ref[...] * jnp.dot(
                a, w_ref[...], preferred_element_type=jnp.float32))

        fo = W_next.shape[1]
        return pl.pallas_call(
            body,
            grid=(GRID,),
            in_specs=[_row_spec(D), _row_spec(D), _row_spec(D), _row_spec(1),
                      _full_spec(1, d), _full_spec(d, fo), _row_spec(1)],
            out_specs=[_row_spec(d), _row_spec(D)],
            out_shape=[jax.ShapeDtypeStruct((N, d), jnp.float32),
                       jax.ShapeDtypeStruct((N, D), jnp.float32)],
        )(p0, p1, hp, dinv, b, W_next, post_dinv)

    if post_dinv is not None:
        def body(p0_ref, p1_ref, hp_ref, dv_ref, b_ref, pdv_ref,
                 a_ref, u_ref):
            ps = (p0_ref[...] + p1_ref[...] + hp_ref[...])[:, :d]
            a = act(dv_ref[...] * ps + b_ref[...])
            a_ref[...] = a
            u_ref[...] = _pad128(pdv_ref[...] * a)

        return pl.pallas_call(
            body,
            grid=(GRID,),
            in_specs=[_row_spec(D), _row_spec(D), _row_spec(D), _row_spec(1),
                      _full_spec(1, d), _row_spec(1)],
            out_specs=[_row_spec(d), _row_spec(D)],
            out_shape=[jax.ShapeDtypeStruct((N, d), jnp.float32),
                       jax.ShapeDtypeStruct((N, D), jnp.float32)],
        )(p0, p1, hp, dinv, b, post_dinv)

    def body(p0_ref, p1_ref, hp_ref, dv_ref, b_ref, a_ref):
        ps = (p0_ref[...] + p1_ref[...] + hp_ref[...])[:, :d]
        a_ref[...] = act(dv_ref[...] * ps + b_ref[...])

    return pl.pallas_call(
        body,
        grid=(GRID,),
        in_specs=[_row_spec(D), _row_spec(D), _row_spec(D), _row_spec(1),
                  _full_spec(1, d)],
        out_specs=_row_spec(d),
        out_shape=jax.ShapeDtypeStruct((N, d), jnp.float32),
    )(p0, p1, hp, dinv, b)


def _deg_combine(p0, p1):
    def body(p0_ref, p1_ref, d1_ref, d2_ref):
        ps = p0_ref[...] + p1_ref[...]
        deg1 = ps[:, 0:1] + 1.0
        deg2 = ps[:, 16:17] + 1.0
        d1_ref[...] = jnp.where(
            deg1 > 0, lax.rsqrt(jnp.maximum(deg1, 1e-12)), 0.0)
        d2_ref[...] = jnp.where(
            deg2 > 0, lax.rsqrt(jnp.maximum(deg2, 1e-12)), 0.0)

    return pl.pallas_call(
        body,
        grid=(GRID,),
        in_specs=[_row_spec(D), _row_spec(D)],
        out_specs=[_row_spec(1), _row_spec(1)],
        out_shape=[jax.ShapeDtypeStruct((N, 1), jnp.float32),
                   jax.ShapeDtypeStruct((N, 1), jnp.float32)],
    )(p0, p1)


def _loop_step(p0, p1, u, xin, xsum, dinv):
    def body(p0_ref, p1_ref, u_ref, x_ref, xs_ref, dv_ref,
             xo_ref, uo_ref, xso_ref):
        xin_v = x_ref[...]
        ps = (p0_ref[...] + p1_ref[...] + u_ref[...])[:, :64]
        xo = dv_ref[...] * ps
        num = jnp.sum(xo * xin_v, axis=1, keepdims=True)
        na = jnp.maximum(
            jnp.sqrt(jnp.sum(xo * xo, axis=1, keepdims=True)), 1e-8)
        nb = jnp.maximum(
            jnp.sqrt(jnp.sum(xin_v * xin_v, axis=1, keepdims=True)), 1e-8)
        w = num / (na * nb)
        xnew = jax.nn.relu(0.5 * (w * xo) + xin_v)
        xo_ref[...] = xnew
        uo_ref[...] = _pad128(dv_ref[...] * xnew)
        xso_ref[...] = xs_ref[...] + xnew

    return pl.pallas_call(
        body,
        grid=(GRID,),
        in_specs=[_row_spec(D), _row_spec(D), _row_spec(D), _row_spec(64),
                  _row_spec(64), _row_spec(1)],
        out_specs=[_row_spec(64), _row_spec(D), _row_spec(64)],
        out_shape=[jax.ShapeDtypeStruct((N, 64), jnp.float32),
                   jax.ShapeDtypeStruct((N, D), jnp.float32),
                   jax.ShapeDtypeStruct((N, 64), jnp.float32)],
    )(p0, p1, u, xin, xsum, dinv)


def _final(t0, t1, g2p, dinv2, b_c2, z_en):
    def body(t0_ref, t1_ref, g_ref, dv_ref, b_ref, z_ref,
             x2_ref, o_ref, q_ref):
        ps = (t0_ref[...] + t1_ref[...] + g_ref[...])[:, :16]
        x2 = jnp.tanh(dv_ref[...] * ps + b_ref[...])
        x2_ref[...] = x2
        o_ref[...] = jax.nn.softmax(x2, axis=1)
        q_ref[...] = jax.nn.softmax(z_ref[...], axis=1)

    return pl.pallas_call(
        body,
        grid=(GRID,),
        in_specs=[_row_spec(D), _row_spec(D), _row_spec(D), _row_spec(1),
                  _full_spec(1, 16), _row_spec(16)],
        out_specs=[_row_spec(16)] * 3,
        out_shape=[jax.ShapeDtypeStruct((N, 16), jnp.float32)] * 3,
    )(t0, t1, g2p, dinv2, b_c2, z_en)


def _identity(v):
    return v


def kernel(x, edge_index, edge_weight, W_enc1, b_enc1, W_enc2, b_enc2, W_enc3,
           b_enc3, W_z, b_z, W_dec1, b_dec1, W_dec2, b_dec2, W_dec3, b_dec3,
           W_xde, b_xde, W_c1, b_c1, W_c2, b_c2):
    srci = edge_index[0].astype(jnp.int32)
    dsti = edge_index[1].astype(jnp.int32)
    src = srci.reshape(NW, SUP, SUBC, K)
    dst = dsti.reshape(NW, SUP, SUBC, K)
    srcu = srci.reshape(NW, SUPU, SUBC, KU)
    dstu = dsti.reshape(NW, SUPU, SUBC, KU)
    wres = edge_weight.astype(jnp.float32).reshape(NW, SUP, SUBC, K)
    relu = jax.nn.relu

    degp = _deg(dst, wres)
    dinv1, dinv2 = _deg_combine(degp[0], degp[1])

    def spmm1(hp):
        p = _spmm(hp, srcu, dstu)
        return p[0], p[1]

    def spmm2(hp, sw):
        f = {16: _spmm_w16, 64: _spmm_w64}[sw]
        p = f(hp, src, dst, wres)
        return p[0], p[1]

    b1 = b_enc1.reshape(1, -1)
    b2 = b_enc2.reshape(1, -1)
    b3 = b_enc3.reshape(1, -1)
    bz = b_z.reshape(1, -1)
    bd1 = b_dec1.reshape(1, -1)
    bd2 = b_dec2.reshape(1, -1)
    bd3 = b_dec3.reshape(1, -1)
    bxde = b_xde.reshape(1, -1)
    bc1 = b_c1.reshape(1, -1)
    bc2 = b_c2.reshape(1, -1)

    h1p = _mm_scale(x, W_enc1, dinv1)
    p0, p1 = spmm1(h1p)
    _, h2p = _combine(p0, p1, h1p, 64, dinv1, b1, relu, W_next=W_enc2,
                      post_dinv=dinv1)
    p0, p1 = spmm1(h2p)
    _, h3p = _combine(p0, p1, h2p, 64, dinv1, b2, relu, W_next=W_enc3,
                      post_dinv=dinv1)
    p0, p1 = spmm1(h3p)
    _, hzp = _combine(p0, p1, h3p, 64, dinv1, b3, relu, W_next=W_z,
                      post_dinv=dinv1)
    p0, p1 = spmm1(hzp)
    z_en, hd1p = _combine(p0, p1, hzp, 16, dinv1, bz, _identity,
                          W_next=W_dec1, post_dinv=dinv1)
    p0, p1 = spmm1(hd1p)
    _, hd2p = _combine(p0, p1, hd1p, 64, dinv1, bd1, relu, W_next=W_dec2,
                       post_dinv=dinv1)
    p0, p1 = spmm1(hd2p)
    _, hd3p = _combine(p0, p1, hd2p, 64, dinv1, bd2, relu, W_next=W_dec3,
                       post_dinv=dinv1)
    p0, p1 = spmm1(hd3p)
    _, hxp = _combine(p0, p1, hd3p, 64, dinv1, bd3, relu, W_next=W_xde,
                      post_dinv=dinv1)
    p0, p1 = spmm1(hxp)
    x_de = _combine(p0, p1, hxp, 128, dinv1, bxde, _identity)

    g1p = _mm_scale(x, W_c1, dinv2)
    t0, t1 = spmm2(g1p, 64)
    xin, u = _combine(t0, t1, g1p, 64, dinv2, bc1, jax.nn.sigmoid,
                      post_dinv=dinv1)
    xsum = jnp.zeros((N, 64), jnp.float32)
    for _ in range(3):
        p0, p1 = spmm1(u)
        xin, u, xsum = _loop_step(p0, p1, u, xin, xsum, dinv1)
    g2p = _mm_scale(xsum, W_c2, dinv2, scale=1.0 / 3.0)
    t0, t1 = spmm2(g2p, 16)
    x_out2, o, q = _final(t0, t1, g2p, dinv2, bc2, z_en)
    return (x_out2, x_de, z_en, q, o)


# speedup vs baseline: 18.7248x; 1.1309x over previous
import functools

import jax
import jax.numpy as jnp
from jax import lax
from jax.experimental import pallas as pl
from jax.experimental.pallas import tpu as pltpu
from jax.experimental.pallas import tpu_sc as plsc

N = 10000
E = 320000
NCORE = 2
NSUB = 16
NW = NCORE * NSUB
EPT = E // NW
K = 80
KU = 125
SUBC = 5
SUBCU = 4
SUP = EPT // (SUBC * K)
SUPU = EPT // (SUBCU * KU)
RPT = 624
SCP = 24
TAIL = N - NSUB * RPT
TAIL_BASE = NSUB * RPT
D = 128

_MESH = plsc.VectorSubcoreMesh(core_axis_name="c", subcore_axis_name="s")


def _make_spmm(weighted, sw=0):
    k = K
    sup = SUP
    scratch = [
        pltpu.VMEM_SHARED((N, D), jnp.float32),
        pltpu.VMEM((2, SUBC, k), jnp.int32),
        pltpu.VMEM((2, SUBC, k), jnp.int32),
        pltpu.VMEM((k, D), jnp.float32),
        pltpu.VMEM((k, D), jnp.float32),
        pltpu.VMEM((SCP, D), jnp.float32),
        pltpu.SemaphoreType.DMA,
        pltpu.SemaphoreType.DMA,
        pltpu.SemaphoreType.DMA,
    ]
    if weighted:
        scratch.append(pltpu.VMEM((2, SUBC, k), jnp.float32))

    @functools.partial(
        pl.kernel,
        mesh=_MESH,
        out_type=jax.ShapeDtypeStruct((NCORE, N, D), jnp.float32),
        scratch_types=scratch,
    )
    def spmm(*refs):
        if weighted:
            (h, srcr, dstr, wr, out, acc, isrc, idst, rows0, rows1, zbuf,
             sem0, sem1, isem, wbuf) = refs
        else:
            (h, srcr, dstr, out, acc, isrc, idst, rows0, rows1, zbuf,
             sem0, sem1, isem) = refs
        bufs = (rows0, rows1)
        sems = (sem0, sem1)
        c = lax.axis_index("c")
        s = lax.axis_index("s")
        wid = s * NCORE + c
        z16 = jnp.zeros((16,), jnp.float32)

        def zrow(r, carry):
            for j in range(D // 16):
                zbuf[r, pl.ds(16 * j, 16)] = z16
            return carry

        lax.fori_loop(0, SCP, zrow, 0)
        base = s * RPT

        def zcp(t, carry):
            off = pl.multiple_of(base + t * SCP, 8)
            pltpu.async_copy(zbuf, acc.at[pl.ds(off, SCP)], sem0)
            return carry

        lax.fori_loop(0, RPT // SCP, zcp, 0)

        @pl.when(s == NSUB - 1)
        def _():
            pltpu.async_copy(zbuf.at[pl.ds(0, TAIL)],
                            acc.at[pl.ds(TAIL_BASE, TAIL)], sem0)

        def zdr(t, carry):
            off = pl.multiple_of(base + t * SCP, 8)
            pltpu.make_async_copy(zbuf, acc.at[pl.ds(off, SCP)], sem0).wait()
            return carry

        lax.fori_loop(0, RPT // SCP, zdr, 0)

        @pl.when(s == NSUB - 1)
        def _():
            pltpu.make_async_copy(zbuf.at[pl.ds(0, TAIL)],
                                  acc.at[pl.ds(TAIL_BASE, TAIL)], sem0).wait()

        pltpu.async_copy(srcr.at[wid, 0], isrc.at[0], isem)
        pltpu.async_copy(dstr.at[wid, 0], idst.at[0], isem)
        if weighted:
            pltpu.async_copy(wr.at[wid, 0], wbuf.at[0], isem)
        plsc.subcore_barrier()

        def outer(t, carry):
            par = t % 2
            pltpu.make_async_copy(srcr.at[wid, t], isrc.at[par], isem).wait()
            pltpu.make_async_copy(dstr.at[wid, t], idst.at[par], isem).wait()
            if weighted:
                pltpu.make_async_copy(wr.at[wid, t], wbuf.at[par], isem).wait()

            @pl.when(t + 1 < sup)
            def _():
                pltpu.async_copy(srcr.at[wid, t + 1], isrc.at[1 - par], isem)
                pltpu.async_copy(dstr.at[wid, t + 1], idst.at[1 - par], isem)
                if weighted:
                    pltpu.async_copy(wr.at[wid, t + 1], wbuf.at[1 - par], isem)

            isr = isrc.at[par]
            ids = idst.at[par]
            handles = [None] * SUBC
            handles[0] = pltpu.async_copy(h.at[isr.at[0]], bufs[0], sems[0])
            for j in range(SUBC):
                if j + 1 < SUBC:
                    handles[j + 1] = pltpu.async_copy(
                        h.at[isr.at[j + 1]], bufs[(j + 1) % 2],
                        sems[(j + 1) % 2])
                handles[j].wait()
                rows = bufs[j % 2]
                if weighted:
                    def scale(g, carry2):
                        st = pl.multiple_of(g * 16, 8)
                        w16 = wbuf[par, j, pl.ds(st, 16)]
                        for b in range(16):
                            wv = w16[b]
                            r = g * 16 + b
                            for q in range(sw // 16):
                                sl = pl.ds(16 * q, 16)
                                rows[r, sl] = rows[r, sl] * wv
                        return carry2

                    lax.fori_loop(0, K // 16, scale, 0)
                pltpu.sync_copy(rows, acc.at[ids.at[j]], add=True)
            return carry

        lax.fori_loop(0, sup, outer, 0)
        plsc.subcore_barrier()

        def wcp(t, carry):
            off = pl.multiple_of(base + t * SCP, 8)
            pltpu.async_copy(acc.at[pl.ds(off, SCP)],
                             out.at[c, pl.ds(off, SCP)], sem0)
            return carry

        lax.fori_loop(0, RPT // SCP, wcp, 0)

        @pl.when(s == NSUB - 1)
        def _():
            pltpu.async_copy(acc.at[pl.ds(TAIL_BASE, TAIL)],
                             out.at[c, pl.ds(TAIL_BASE, TAIL)], sem0)

        def wdr(t, carry):
            off = pl.multiple_of(base + t * SCP, 8)
            pltpu.make_async_copy(acc.at[pl.ds(off, SCP)],
                                  out.at[c, pl.ds(off, SCP)], sem0).wait()
            return carry

        lax.fori_loop(0, RPT // SCP, wdr, 0)

        @pl.when(s == NSUB - 1)
        def _():
            pltpu.make_async_copy(acc.at[pl.ds(TAIL_BASE, TAIL)],
                                  out.at[c, pl.ds(TAIL_BASE, TAIL)],
                                  sem0).wait()

    return spmm


def _make_spmm_plain():
    scratch = [
        pltpu.VMEM_SHARED((N, D), jnp.float32),
        pltpu.VMEM((2, SUBCU, KU), jnp.int32),
        pltpu.VMEM((2, SUBCU, KU), jnp.int32),
        pltpu.VMEM((KU, D), jnp.float32),
        pltpu.VMEM((KU, D), jnp.float32),
        pltpu.VMEM((SCP, D), jnp.float32),
        pltpu.SemaphoreType.DMA,
        pltpu.SemaphoreType.DMA,
        pltpu.SemaphoreType.DMA,
        pltpu.SemaphoreType.DMA,
        pltpu.SemaphoreType.DMA,
    ]

    @functools.partial(
        pl.kernel,
        mesh=_MESH,
        out_type=jax.ShapeDtypeStruct((NCORE, N, D), jnp.float32),
        scratch_types=scratch,
    )
    def spmm(h, srcr, dstr, out, acc, isrc, idst, rows0, rows1, zbuf,
             gsem0, gsem1, ssem0, ssem1, isem):
        bufs = (rows0, rows1)
        gsems = (gsem0, gsem1)
        ssems = (ssem0, ssem1)
        c = lax.axis_index("c")
        s = lax.axis_index("s")
        wid = s * NCORE + c
        z16 = jnp.zeros((16,), jnp.float32)

        def zrow(r, carry):
            for j in range(D // 16):
                zbuf[r, pl.ds(16 * j, 16)] = z16
            return carry

        lax.fori_loop(0, SCP, zrow, 0)
        base = s * RPT

        def zcp(t, carry):
            off = pl.multiple_of(base + t * SCP, 8)
            pltpu.async_copy(zbuf, acc.at[pl.ds(off, SCP)], gsem0)
            return carry

        lax.fori_loop(0, RPT // SCP, zcp, 0)

        @pl.when(s == NSUB - 1)
        def _():
            pltpu.async_copy(zbuf.at[pl.ds(0, TAIL)],
                             acc.at[pl.ds(TAIL_BASE, TAIL)], gsem0)

        def zdr(t, carry):
            off = pl.multiple_of(base + t * SCP, 8)
            pltpu.make_async_copy(zbuf, acc.at[pl.ds(off, SCP)], gsem0).wait()
            return carry

        lax.fori_loop(0, RPT // SCP, zdr, 0)

        @pl.when(s == NSUB - 1)
        def _():
            pltpu.make_async_copy(zbuf.at[pl.ds(0, TAIL)],
                                  acc.at[pl.ds(TAIL_BASE, TAIL)],
                                  gsem0).wait()

        pltpu.sync_copy(srcr.at[wid, 0], isrc.at[0])
        pltpu.sync_copy(dstr.at[wid, 0], idst.at[0])
        plsc.subcore_barrier()
        pltpu.async_copy(h.at[isrc.at[0].at[0]], bufs[0], gsems[0])

        def outer(t, carry):
            par = t % 2
            isr = isrc.at[par]
            ids = idst.at[par]
            for j in range(SUBCU):
                pb = j % 2
                nb = (j + 1) % 2
                if j == 1:
                    @pl.when(t + 1 < SUPU)
                    def _():
                        pltpu.async_copy(srcr.at[wid, t + 1],
                                         isrc.at[1 - par], isem)
                        pltpu.async_copy(dstr.at[wid, t + 1],
                                         idst.at[1 - par], isem)
                if j < SUBCU - 1:
                    if j == 0:
                        @pl.when(t > 0)
                        def _():
                            pltpu.make_async_copy(
                                bufs[nb], acc.at[ids.at[j]], ssems[nb]).wait()
                    else:
                        pltpu.make_async_copy(
                            bufs[nb], acc.at[ids.at[j]], ssems[nb]).wait()
                    pltpu.async_copy(h.at[isr.at[j + 1]], bufs[nb], gsems[nb])
                else:
                    @pl.when(t + 1 < SUPU)
                    def _():
                        pltpu.make_async_copy(
                            srcr.at[wid, t + 1], isrc.at[1 - par],
                            isem).wait()
                        pltpu.make_async_copy(
                            dstr.at[wid, t + 1], idst.at[1 - par],
                            isem).wait()
                        pltpu.make_async_copy(
                            bufs[nb], acc.at[ids.at[j]], ssems[nb]).wait()
                        pltpu.async_copy(h.at[isrc.at[1 - par].at[0]],
                                         bufs[nb], gsems[nb])
                pltpu.make_async_copy(h.at[isr.at[j]], bufs[pb],
                                      gsems[pb]).wait()
                pltpu.async_copy(bufs[pb], acc.at[ids.at[j]], ssems[pb],
                                 add=True)
            return carry

        lax.fori_loop(0, SUPU, outer, 0)
        pltpu.make_async_copy(bufs[0], acc.at[idst.at[0].at[0]],
                              ssems[0]).wait()
        pltpu.make_async_copy(bufs[1], acc.at[idst.at[0].at[1]],
                              ssems[1]).wait()
        plsc.subcore_barrier()

        def wcp(t, carry):
            off = pl.multiple_of(base + t * SCP, 8)
            pltpu.async_copy(acc.at[pl.ds(off, SCP)],
                             out.at[c, pl.ds(off, SCP)], gsem0)
            return carry

        lax.fori_loop(0, RPT // SCP, wcp, 0)

        @pl.when(s == NSUB - 1)
        def _():
            pltpu.async_copy(acc.at[pl.ds(TAIL_BASE, TAIL)],
                             out.at[c, pl.ds(TAIL_BASE, TAIL)], gsem0)

        def wdr(t, carry):
            off = pl.multiple_of(base + t * SCP, 8)
            pltpu.make_async_copy(acc.at[pl.ds(off, SCP)],
                                  out.at[c, pl.ds(off, SCP)], gsem0).wait()
            return carry

        lax.fori_loop(0, RPT // SCP, wdr, 0)

        @pl.when(s == NSUB - 1)
        def _():
            pltpu.make_async_copy(acc.at[pl.ds(TAIL_BASE, TAIL)],
                                  out.at[c, pl.ds(TAIL_BASE, TAIL)],
                                  gsem0).wait()

    return spmm


def _make_deg():
    @functools.partial(
        pl.kernel,
        mesh=_MESH,
        out_type=jax.ShapeDtypeStruct((NCORE, N, D), jnp.float32),
        scratch_types=[
            pltpu.VMEM_SHARED((N, D), jnp.float32),
            pltpu.VMEM((SUBC, K), jnp.int32),
            pltpu.VMEM((SUBC, K), jnp.float32),
            pltpu.VMEM((K, D), jnp.float32),
            pltpu.VMEM((SCP, D), jnp.float32),
        ],
    )
    def deg(dstr, wr, out, acc, idst, wbuf, rows, zbuf):
        c = lax.axis_index("c")
        s = lax.axis_index("s")
        wid = s * NCORE + c
        z16 = jnp.zeros((16,), jnp.float32)
        one16 = jnp.ones((16,), jnp.float32)

        def zrow(r, carry):
            for j in range(D // 16):
                zbuf[r, pl.ds(16 * j, 16)] = z16
            return carry

        lax.fori_loop(0, SCP, zrow, 0)

        def orow(r, carry):
            rows[r, pl.ds(0, 16)] = one16
            for j in range(2, D // 16):
                rows[r, pl.ds(16 * j, 16)] = z16
            return carry

        lax.fori_loop(0, K, orow, 0)
        base = s * RPT

        def zcp(t, carry):
            off = pl.multiple_of(base + t * SCP, 8)
            pltpu.sync_copy(zbuf, acc.at[pl.ds(off, SCP)])
            return carry

        lax.fori_loop(0, RPT // SCP, zcp, 0)

        @pl.when(s == NSUB - 1)
        def _():
            pltpu.sync_copy(zbuf.at[pl.ds(0, TAIL)],
                            acc.at[pl.ds(TAIL_BASE, TAIL)])

        plsc.subcore_barrier()

        def outer(t, carry):
            pltpu.sync_copy(dstr.at[wid, t], idst)
            pltpu.sync_copy(wr.at[wid, t], wbuf)
            for j in range(SUBC):
                def fill(g, carry2):
                    st = pl.multiple_of(g * 16, 8)
                    w16 = wbuf[j, pl.ds(st, 16)]
                    for b in range(16):
                        rows[g * 16 + b, pl.ds(16, 16)] = one16 * w16[b]
                    return carry2

                lax.fori_loop(0, K // 16, fill, 0)
                pltpu.sync_copy(rows, acc.at[idst.at[j]], add=True)
            return carry

        lax.fori_loop(0, SUP, outer, 0)
        plsc.subcore_barrier()

        def wcp(t, carry):
            off = pl.multiple_of(base + t * SCP, 8)
            pltpu.sync_copy(acc.at[pl.ds(off, SCP)],
                            out.at[c, pl.ds(off, SCP)])
            return carry

        lax.fori_loop(0, RPT // SCP, wcp, 0)

        @pl.when(s == NSUB - 1)
        def _():
            pltpu.sync_copy(acc.at[pl.ds(TAIL_BASE, TAIL)],
                            out.at[c, pl.ds(TAIL_BASE, TAIL)])

    return deg


_spmm = _make_spmm_plain()
_spmm_w64 = _make_spmm(True, 64)
_spmm_w16 = _make_spmm(True, 16)
_deg = _make_deg()


BR = 1000
GRID = N // BR


def _row_spec(d):
    return pl.BlockSpec((BR, d), lambda i: (i, 0))


def _full_spec(a, b):
    return pl.BlockSpec((a, b), lambda i: (0, 0))


def _pad128(a):
    fo = a.shape[1]
    if fo == D:
        return a
    return jnp.concatenate(
        [a, jnp.zeros((a.shape[0], D - fo), jnp.float32)], axis=1)


def _mm_scale(x, W, dinv, scale=1.0):
    fi, fo = W.shape

    def body(x_ref, w_ref, dv_ref, o_ref):
        acc = jnp.dot(x_ref[...], w_ref[...],
                      preferred_element_type=jnp.float32)
        o_ref[...] = _pad128((dv_ref[...] * scale) * acc)

    return pl.pallas_call(
        body,
        grid=(GRID,),
        in_specs=[_row_spec(fi), _full_spec(fi, fo), _row_spec(1)],
        out_specs=_row_spec(D),
        out_shape=jax.ShapeDtypeStruct((N, D), jnp.float32),
    )(x, W, dinv)


def _combine(p0, p1, hp, d, dinv, b, act, W_next=None, post_dinv=None):
    if W_next is not None:
        def body(p0_ref, p1_ref, hp_ref, dv_ref, b_ref, w_ref, pdv_ref,
                 a_ref, h_ref):
            ps = (p0_ref[...] + p1_ref[...] + hp_ref[...])[:, :d]
            a = act(dv_ref[...] * ps + b_ref[...])
            a_ref[...] = a
            h_ref[...] = _pad128(pdv_ref[...] * jnp.dot(
                a, w_ref[...], preferred_element_type=jnp.float32))

        fo = W_next.shape[1]
        return pl.pallas_call(
            body,
            grid=(GRID,),
            in_specs=[_row_spec(D), _row_spec(D), _row_spec(D), _row_spec(1),
                      _full_spec(1, d), _full_spec(d, fo), _row_spec(1)],
            out_specs=[_row_spec(d), _row_spec(D)],
            out_shape=[jax.ShapeDtypeStruct((N, d), jnp.float32),
                       jax.ShapeDtypeStruct((N, D), jnp.float32)],
        )(p0, p1, hp, dinv, b, W_next, post_dinv)

    if post_dinv is not None:
        def body(p0_ref, p1_ref, hp_ref, dv_ref, b_ref, pdv_ref,
                 a_ref, u_ref):
            ps = (p0_ref[...] + p1_ref[...] + hp_ref[...])[:, :d]
            a = act(dv_ref[...] * ps + b_ref[...])
            a_ref[...] = a
            u_ref[...] = _pad128(pdv_ref[...] * a)

        return pl.pallas_call(
            body,
            grid=(GRID,),
            in_specs=[_row_spec(D), _row_spec(D), _row_spec(D), _row_spec(1),
                      _full_spec(1, d), _row_spec(1)],
            out_specs=[_row_spec(d), _row_spec(D)],
            out_shape=[jax.ShapeDtypeStruct((N, d), jnp.float32),
                       jax.ShapeDtypeStruct((N, D), jnp.float32)],
        )(p0, p1, hp, dinv, b, post_dinv)

    def body(p0_ref, p1_ref, hp_ref, dv_ref, b_ref, a_ref):
        ps = (p0_ref[...] + p1_ref[...] + hp_ref[...])[:, :d]
        a_ref[...] = act(dv_ref[...] * ps + b_ref[...])

    return pl.pallas_call(
        body,
        grid=(GRID,),
        in_specs=[_row_spec(D), _row_spec(D), _row_spec(D), _row_spec(1),
                  _full_spec(1, d)],
        out_specs=_row_spec(d),
        out_shape=jax.ShapeDtypeStruct((N, d), jnp.float32),
    )(p0, p1, hp, dinv, b)


def _deg_combine(p0, p1):
    def body(p0_ref, p1_ref, d1_ref, d2_ref):
        ps = p0_ref[...] + p1_ref[...]
        deg1 = ps[:, 0:1] + 1.0
        deg2 = ps[:, 16:17] + 1.0
        d1_ref[...] = jnp.where(
            deg1 > 0, lax.rsqrt(jnp.maximum(deg1, 1e-12)), 0.0)
        d2_ref[...] = jnp.where(
            deg2 > 0, lax.rsqrt(jnp.maximum(deg2, 1e-12)), 0.0)

    return pl.pallas_call(
        body,
        grid=(GRID,),
        in_specs=[_row_spec(D), _row_spec(D)],
        out_specs=[_row_spec(1), _row_spec(1)],
        out_shape=[jax.ShapeDtypeStruct((N, 1), jnp.float32),
                   jax.ShapeDtypeStruct((N, 1), jnp.float32)],
    )(p0, p1)


def _loop_step(p0, p1, u, xin, xsum, dinv):
    def body(p0_ref, p1_ref, u_ref, x_ref, xs_ref, dv_ref,
             xo_ref, uo_ref, xso_ref):
        xin_v = x_ref[...]
        ps = (p0_ref[...] + p1_ref[...] + u_ref[...])[:, :64]
        xo = dv_ref[...] * ps
        num = jnp.sum(xo * xin_v, axis=1, keepdims=True)
        na = jnp.maximum(
            jnp.sqrt(jnp.sum(xo * xo, axis=1, keepdims=True)), 1e-8)
        nb = jnp.maximum(
            jnp.sqrt(jnp.sum(xin_v * xin_v, axis=1, keepdims=True)), 1e-8)
        w = num / (na * nb)
        xnew = jax.nn.relu(0.5 * (w * xo) + xin_v)
        xo_ref[...] = xnew
        uo_ref[...] = _pad128(dv_ref[...] * xnew)
        xso_ref[...] = xs_ref[...] + xnew

    return pl.pallas_call(
        body,
        grid=(GRID,),
        in_specs=[_row_spec(D), _row_spec(D), _row_spec(D), _row_spec(64),
                  _row_spec(64), _row_spec(1)],
        out_specs=[_row_spec(64), _row_spec(D), _row_spec(64)],
        out_shape=[jax.ShapeDtypeStruct((N, 64), jnp.float32),
                   jax.ShapeDtypeStruct((N, D), jnp.float32),
                   jax.ShapeDtypeStruct((N, 64), jnp.float32)],
    )(p0, p1, u, xin, xsum, dinv)


def _final(t0, t1, g2p, dinv2, b_c2, z_en):
    def body(t0_ref, t1_ref, g_ref, dv_ref, b_ref, z_ref,
             x2_ref, o_ref, q_ref):
        ps = (t0_ref[...] + t1_ref[...] + g_ref[...])[:, :16]
        x2 = jnp.tanh(dv_ref[...] * ps + b_ref[...])
        x2_ref[...] = x2
        o_ref[...] = jax.nn.softmax(x2, axis=1)
        q_ref[...] = jax.nn.softmax(z_ref[...], axis=1)

    return pl.pallas_call(
        body,
        grid=(GRID,),
        in_specs=[_row_spec(D), _row_spec(D), _row_spec(D), _row_spec(1),
                  _full_spec(1, 16), _row_spec(16)],
        out_specs=[_row_spec(16)] * 3,
        out_shape=[jax.ShapeDtypeStruct((N, 16), jnp.float32)] * 3,
    )(t0, t1, g2p, dinv2, b_c2, z_en)


def _identity(v):
    return v


def kernel(x, edge_index, edge_weight, W_enc1, b_enc1, W_enc2, b_enc2, W_enc3,
           b_enc3, W_z, b_z, W_dec1, b_dec1, W_dec2, b_dec2, W_dec3, b_dec3,
           W_xde, b_xde, W_c1, b_c1, W_c2, b_c2):
    srci = edge_index[0].astype(jnp.int32)
    dsti = edge_index[1].astype(jnp.int32)
    src = srci.reshape(NW, SUP, SUBC, K)
    dst = dsti.reshape(NW, SUP, SUBC, K)
    srcu = srci.reshape(NW, SUPU, SUBCU, KU)
    dstu = dsti.reshape(NW, SUPU, SUBCU, KU)
    wres = edge_weight.astype(jnp.float32).reshape(NW, SUP, SUBC, K)
    relu = jax.nn.relu

    degp = _deg(dst, wres)
    dinv1, dinv2 = _deg_combine(degp[0], degp[1])

    def spmm1(hp):
        p = _spmm(hp, srcu, dstu)
        return p[0], p[1]

    def spmm2(hp, sw):
        f = {16: _spmm_w16, 64: _spmm_w64}[sw]
        p = f(hp, src, dst, wres)
        return p[0], p[1]

    b1 = b_enc1.reshape(1, -1)
    b2 = b_enc2.reshape(1, -1)
    b3 = b_enc3.reshape(1, -1)
    bz = b_z.reshape(1, -1)
    bd1 = b_dec1.reshape(1, -1)
    bd2 = b_dec2.reshape(1, -1)
    bd3 = b_dec3.reshape(1, -1)
    bxde = b_xde.reshape(1, -1)
    bc1 = b_c1.reshape(1, -1)
    bc2 = b_c2.reshape(1, -1)

    h1p = _mm_scale(x, W_enc1, dinv1)
    p0, p1 = spmm1(h1p)
    _, h2p = _combine(p0, p1, h1p, 64, dinv1, b1, relu, W_next=W_enc2,
                      post_dinv=dinv1)
    p0, p1 = spmm1(h2p)
    _, h3p = _combine(p0, p1, h2p, 64, dinv1, b2, relu, W_next=W_enc3,
                      post_dinv=dinv1)
    p0, p1 = spmm1(h3p)
    _, hzp = _combine(p0, p1, h3p, 64, dinv1, b3, relu, W_next=W_z,
                      post_dinv=dinv1)
    p0, p1 = spmm1(hzp)
    z_en, hd1p = _combine(p0, p1, hzp, 16, dinv1, bz, _identity,
                          W_next=W_dec1, post_dinv=dinv1)
    p0, p1 = spmm1(hd1p)
    _, hd2p = _combine(p0, p1, hd1p, 64, dinv1, bd1, relu, W_next=W_dec2,
                       post_dinv=dinv1)
    p0, p1 = spmm1(hd2p)
    _, hd3p = _combine(p0, p1, hd2p, 64, dinv1, bd2, relu, W_next=W_dec3,
                       post_dinv=dinv1)
    p0, p1 = spmm1(hd3p)
    _, hxp = _combine(p0, p1, hd3p, 64, dinv1, bd3, relu, W_next=W_xde,
                      post_dinv=dinv1)
    p0, p1 = spmm1(hxp)
    x_de = _combine(p0, p1, hxp, 128, dinv1, bxde, _identity)

    g1p = _mm_scale(x, W_c1, dinv2)
    t0, t1 = spmm2(g1p, 64)
    xin, u = _combine(t0, t1, g1p, 64, dinv2, bc1, jax.nn.sigmoid,
                      post_dinv=dinv1)
    xsum = jnp.zeros((N, 64), jnp.float32)
    for _ in range(3):
        p0, p1 = spmm1(u)
        xin, u, xsum = _loop_step(p0, p1, u, xin, xsum, dinv1)
    g2p = _mm_scale(xsum, W_c2, dinv2, scale=1.0 / 3.0)
    t0, t1 = spmm2(g2p, 16)
    x_out2, o, q = _final(t0, t1, g2p, dinv2, bc2, z_en)
    return (x_out2, x_de, z_en, q, o)
